# baseline (speedup 1.0000x reference)
import functools

import jax
import jax.numpy as jnp
from jax import lax
from jax.experimental import pallas as pl
from jax.experimental.pallas import tpu as pltpu
from jax.experimental.pallas import tpu_sc as plsc

N = 10000
E = 320000
K = 32
NHID = 128
SDIM = 32
NPG = 16
NCLASS = 16
NGRAPH = 128
NP_ = 10240
NAK = NP_ * K
NC, NS = 2, 16
ROWS = NP_ // NS
CH = 80
ECH = 64
TCHUNK = 320
EPT = TCHUNK * ECH
EP = NS * EPT
EPC = E // NC
DPT = EPC // NS
DCHUNK = DPT // CH
GPT = NAK // (NC * NS)
GCH = 128
GSTAGE = 2048
_P = lax.Precision.HIGHEST
_MESH = plsc.VectorSubcoreMesh(core_axis_name="c", subcore_axis_name="s",
                               num_cores=NC, num_subcores=NS)
_SC_LINEAR = pltpu.CompilerParams(use_tc_tiling_on_sc=False)


def _dot(a, b):
    return jnp.dot(a, b, preferred_element_type=jnp.float32, precision=_P)


def _deg_body(dst_hbm, ones_hbm, zeros_hbm, out_hbm, ones_v, didx, acc_sh):
    c = lax.axis_index("c")
    s = lax.axis_index("s")
    pltpu.sync_copy(ones_hbm, ones_v)
    pltpu.sync_copy(zeros_hbm, acc_sh.at[pl.ds(s * ROWS, ROWS)])
    plsc.subcore_barrier()
    base0 = c * EPC + s * DPT

    def chunk(g, carry):
        pltpu.sync_copy(dst_hbm.at[pl.ds(base0 + g * CH, CH)], didx)
        pltpu.sync_copy(ones_v, acc_sh.at[didx], add=True)
        return carry

    lax.fori_loop(0, DCHUNK, chunk, 0)
    plsc.subcore_barrier()
    pltpu.sync_copy(acc_sh.at[pl.ds(s * ROWS, ROWS)],
                    out_hbm.at[c, pl.ds(s * ROWS, ROWS)])


_deg_kernel = pl.kernel(
    _deg_body,
    out_type=jax.ShapeDtypeStruct((NC, NP_, 16), jnp.float32),
    mesh=_MESH,
    scratch_types=[
        pltpu.VMEM((CH, 16), jnp.float32),
        pltpu.VMEM((CH,), jnp.int32),
        pltpu.VMEM_SHARED((NP_, 16), jnp.float32),
    ],
    compiler_params=_SC_LINEAR,
)


def _gather_body(g_hbm, idx_hbm, out_hbm, idx_v, stage_v, sem):
    wid = lax.axis_index("s") * NC + lax.axis_index("c")
    base0 = wid * GPT

    def outer(o, carry):
        def inner(j, carry2):
            pltpu.sync_copy(
                idx_hbm.at[pl.ds(base0 + o * GSTAGE + j * GCH, GCH)], idx_v)
            pltpu.async_copy(g_hbm.at[idx_v],
                             stage_v.at[pl.ds(j * GCH, GCH)], sem).wait()
            return carry2

        lax.fori_loop(0, GSTAGE // GCH, inner, 0)
        pltpu.sync_copy(stage_v,
                        out_hbm.at[pl.ds(base0 + o * GSTAGE, GSTAGE)])
        return carry

    lax.fori_loop(0, GPT // GSTAGE, outer, 0)


_gather_kernel = pl.kernel(
    _gather_body,
    out_type=jax.ShapeDtypeStruct((NAK, 16), jnp.float32),
    mesh=_MESH,
    scratch_types=[
        pltpu.VMEM((GCH,), jnp.int32),
        pltpu.VMEM((GSTAGE, 16), jnp.float32),
        pltpu.SemaphoreType.DMA,
    ],
    compiler_params=_SC_LINEAR,
)


_NBUF = 4


def _agg_body(y_hbm, tc_hbm, src_hbm, dst_hbm, zeros_hbm, out_hbm,
              *scr):
    sidx = scr[0:_NBUF]
    didx = scr[_NBUF:2 * _NBUF]
    bufs = scr[2 * _NBUF:3 * _NBUF]
    acc_sh = scr[3 * _NBUF]
    gsem = scr[3 * _NBUF + 1:4 * _NBUF + 1]
    isem = scr[4 * _NBUF + 1:5 * _NBUF + 1]
    c = lax.axis_index("c")
    s = lax.axis_index("s")
    base0 = s * EPT

    def iload(g, k):
        pltpu.async_copy(src_hbm.at[pl.ds(base0 + g * ECH, ECH)],
                         sidx[k], isem[k])
        pltpu.async_copy(dst_hbm.at[pl.ds(base0 + g * ECH, ECH)],
                         didx[k], isem[k])

    def iwait(g, k):
        pltpu.make_async_copy(
            src_hbm.at[pl.ds(base0 + g * ECH, ECH)], sidx[k], isem[k]).wait()
        pltpu.make_async_copy(
            dst_hbm.at[pl.ds(base0 + g * ECH, ECH)], didx[k], isem[k]).wait()

    def gstart(k):
        @pl.when(c == 0)
        def _():
            pltpu.async_copy(y_hbm.at[sidx[k]], bufs[k], gsem[k])

        @pl.when(c == 1)
        def _():
            pltpu.async_copy(tc_hbm.at[sidx[k]], bufs[k], gsem[k])

    def gwait(k):
        pltpu.make_async_copy(y_hbm.at[sidx[k]], bufs[k], gsem[k]).wait()

    def scat(k):
        pltpu.sync_copy(bufs[k], acc_sh.at[didx[k]], add=True)

    for k in range(_NBUF):
        iload(k, k)
    pltpu.sync_copy(zeros_hbm, acc_sh.at[pl.ds(s * ROWS, ROWS)])
    plsc.subcore_barrier()
    for k in range(_NBUF):
        iwait(k, k)
        gstart(k)

    def quad(i, carry):
        g = _NBUF * i
        for k in range(_NBUF):
            gwait(k)
            scat(k)
            iload(g + k + _NBUF, k)
            iwait(g + k + _NBUF, k)
            gstart(k)
        return carry

    lax.fori_loop(0, TCHUNK // _NBUF, quad, 0)
    for k in range(_NBUF):
        gwait(k)
    plsc.subcore_barrier()
    pltpu.sync_copy(acc_sh.at[pl.ds(s * ROWS, ROWS)],
                    out_hbm.at[c, pl.ds(s * ROWS, ROWS)])


_agg_kernel = pl.kernel(
    _agg_body,
    out_type=jax.ShapeDtypeStruct((NC, NP_, NHID), jnp.float32),
    mesh=_MESH,
    scratch_types=(
        [pltpu.VMEM((ECH,), jnp.int32)] * (2 * _NBUF)
        + [pltpu.VMEM((ECH, NHID), jnp.float32)] * _NBUF
        + [pltpu.VMEM_SHARED((NP_, NHID), jnp.float32)]
        + [pltpu.SemaphoreType.DMA] * (2 * _NBUF)
    ),
)


B = 1024
GRID = NP_ // B


def _tc1_body(x_ref, dm_ref, wpre_ref, bpre_ref, wa_ref, wb_ref, hidb_ref,
              dc1_ref, db1_ref, dc2_ref, db2_ref,
              h_ref, g_ref, c_ref, d_ref):
    h = _dot(x_ref[...], wpre_ref[...]) + bpre_ref[...]
    h_ref[...] = h
    g_ref[...] = _dot(h, wa_ref[...])
    c_ref[...] = _dot(h, wb_ref[...]) + hidb_ref[...]
    dm = dm_ref[...]
    d = jnp.zeros_like(dm) + db2_ref[0, 0]
    for p in range(NPG):
        d = d + jax.nn.relu(dm * dc1_ref[0, p] + db1_ref[0, p]) * dc2_ref[0, p]
    d_ref[...] = d


def _tc1(xp, dmp, w_pre, b_pre, wa, wb, hid_b, dc_w1, dc_b1, dc_w2, dc_b2):
    full = lambda shp: pl.BlockSpec(shp, lambda i: (0, 0))
    smem = lambda shp: pl.BlockSpec(shp, lambda i: (0, 0),
                                    memory_space=pltpu.SMEM)
    row = lambda w: pl.BlockSpec((B, w), lambda i: (i, 0))
    return pl.pallas_call(
        _tc1_body,
        grid=(GRID,),
        in_specs=[row(128), row(K), full((128, 128)), full((1, 128)),
                  full((128, NPG)), full((128, NPG)), full((1, NPG)),
                  smem((1, NPG)), smem((1, NPG)), smem((1, NPG)),
                  smem((1, 1))],
        out_specs=[row(128), row(NPG), row(NPG), row(K)],
        out_shape=[jax.ShapeDtypeStruct((NP_, 128), jnp.float32),
                   jax.ShapeDtypeStruct((NP_, NPG), jnp.float32),
                   jax.ShapeDtypeStruct((NP_, NPG), jnp.float32),
                   jax.ShapeDtypeStruct((NP_, K), jnp.float32)],
    )(xp, dmp, w_pre, b_pre, wa, wb, hid_b, dc_w1, dc_b1, dc_w2, dc_b2)


def _tc2_body(sub_ref, d_ref, c_ref, stc_ref, eswa_ref, eswb_ref, esb_ref,
              s_ref):
    kk = lax.broadcasted_iota(jnp.int32, (K, K * NPG), 0)
    mm = lax.broadcasted_iota(jnp.int32, (K, K * NPG), 1)
    e32 = (kk == mm // NPG).astype(jnp.float32)
    jj = lax.broadcasted_iota(jnp.int32, (NPG, K * NPG), 0)
    m2 = lax.broadcasted_iota(jnp.int32, (NPG, K * NPG), 1)
    e16 = (jj == m2 % NPG).astype(jnp.float32)
    d_exp = _dot(d_ref[...], e32)
    c_t = _dot(c_ref[...], e16)
    msgs = jax.nn.relu(d_exp * sub_ref[...] + c_t)
    x1 = _dot(msgs, e16.T) * (1.0 / K)
    s_ref[...] = (_dot(stc_ref[...], eswa_ref[...])
                  + _dot(x1, eswb_ref[...]) + esb_ref[...])


def _tc2(sub2d, dp, cp, stcp, eswa, eswb, es_b):
    full = lambda shp: pl.BlockSpec(shp, lambda i: (0, 0))
    row = lambda w: pl.BlockSpec((B, w), lambda i: (i, 0))
    return pl.pallas_call(
        _tc2_body,
        grid=(GRID,),
        in_specs=[row(K * NPG), row(K), row(NPG), row(SDIM),
                  full((SDIM, 128)), full((NPG, 128)), full((1, 128))],
        out_specs=row(128),
        out_shape=jax.ShapeDtypeStruct((NP_, 128), jnp.float32),
    )(sub2d, dp, cp, stcp, eswa, eswb, es_b)


def _tc3a_body(hx_ref, s_ref, acc_ref, w1a_ref, w1b_ref, gcnw_ref,
               y_ref, tc_ref):
    y = _dot(hx_ref[...], w1a_ref[...]) + _dot(s_ref[...], w1b_ref[...])
    hs = _dot(s_ref[...], gcnw_ref[...])
    deg = acc_ref[0, :, 0:1] + acc_ref[1, :, 0:1] + 1.0
    dis = lax.rsqrt(deg)
    y_ref[...] = y
    tc_ref[...] = dis * hs


def _tc3a(hx, s, acc16, w1a, w1b, gcnw):
    full = lambda shp: pl.BlockSpec(shp, lambda i: (0, 0))
    row = lambda w: pl.BlockSpec((B, w), lambda i: (i, 0))
    pair = lambda w: pl.BlockSpec((NC, B, w), lambda i: (0, i, 0))
    return pl.pallas_call(
        _tc3a_body,
        grid=(GRID,),
        in_specs=[row(128), row(128), pair(16),
                  full((128, 128)), full((128, 128)), full((128, 128))],
        out_specs=[row(128), row(128)],
        out_shape=[jax.ShapeDtypeStruct((NP_, 128), jnp.float32),
                   jax.ShapeDtypeStruct((NP_, 128), jnp.float32)],
    )(hx, s, acc16, w1a, w1b, gcnw)


def _tc3b_body(y_ref, tc_ref, agg_ref, acc_ref, b1_ref, w2_ref, b2_ref,
               gcnb_ref, hx_ref, s_ref):
    deg = acc_ref[0, :, 0:1] + acc_ref[1, :, 0:1] + 1.0
    dis = lax.rsqrt(deg)
    hg = _dot(jax.nn.relu(y_ref[...] + agg_ref[0] + b1_ref[...]),
              w2_ref[...]) + b2_ref[...]
    hx_ref[...] = jax.nn.relu(hg)
    s_ref[...] = jnp.tanh(dis * (agg_ref[1] + tc_ref[...]) + gcnb_ref[...])


def _tc3b(y, tcs, agg, acc16, b1, w2, b2, gcnb):
    full = lambda shp: pl.BlockSpec(shp, lambda i: (0, 0))
    row = lambda w: pl.BlockSpec((B, w), lambda i: (i, 0))
    pair = lambda w: pl.BlockSpec((NC, B, w), lambda i: (0, i, 0))
    return pl.pallas_call(
        _tc3b_body,
        grid=(GRID,),
        in_specs=[row(128), row(128), pair(128), pair(16), full((1, 128)),
                  full((128, 128)), full((1, 128)), full((1, 128))],
        out_specs=[row(128), row(128)],
        out_shape=[jax.ShapeDtypeStruct((NP_, 128), jnp.float32),
                   jax.ShapeDtypeStruct((NP_, 128), jnp.float32)],
    )(y, tcs, agg, acc16, b1, w2, b2, gcnb)


def _tc4_body(hx_ref, s_ref, batch_ref, wa_ref, wb_ref, whpb_ref, out_ref):
    i = pl.program_id(0)
    hx2 = (_dot(hx_ref[...], wa_ref[...]) + _dot(s_ref[...], wb_ref[...])
           + whpb_ref[...])
    gids = lax.broadcasted_iota(jnp.int32, (NGRAPH, B), 0)
    mask = (gids == batch_ref[0]).astype(jnp.float32)
    part = _dot(mask, hx2)

    @pl.when(i == 0)
    def _():
        out_ref[...] = part

    @pl.when(i > 0)
    def _():
        out_ref[...] = out_ref[...] + part


def _tc4(hx, s, batch2d, wa, wb, whp_b):
    full = lambda shp: pl.BlockSpec(shp, lambda i: (0, 0))
    row = lambda w: pl.BlockSpec((B, w), lambda i: (i, 0))
    return pl.pallas_call(
        _tc4_body,
        grid=(GRID,),
        in_specs=[row(128), row(128),
                  pl.BlockSpec((1, 1, B), lambda i: (i, 0, 0)),
                  full((128, 128)), full((128, 128)), full((1, 128))],
        out_specs=full((NGRAPH, 128)),
        out_shape=jax.ShapeDtypeStruct((NGRAPH, 128), jnp.float32),
    )(hx, s, batch2d, wa, wb, whp_b)


def _tc5_body(pool_ref, pw_ref, pb_ref, rw_ref, rb_ref, out_ref):
    p = jax.nn.relu(_dot(pool_ref[...], pw_ref[...]) + pb_ref[...])
    lg = _dot(p, rw_ref[...]) + rb_ref[...]
    m = jnp.max(lg, axis=1, keepdims=True)
    e = lg - m
    out_ref[...] = e - jnp.log(jnp.sum(jnp.exp(e), axis=1, keepdims=True))


def _tc5(pooled, post_w, post_b, ro_w, ro_b):
    full = lambda shp: pl.BlockSpec(shp, lambda i: (0, 0))
    return pl.pallas_call(
        _tc5_body,
        grid=(1,),
        in_specs=[full((NGRAPH, 128)), full((128, 128)), full((1, 128)),
                  full((128, NCLASS)), full((1, NCLASS))],
        out_specs=full((NGRAPH, NCLASS)),
        out_shape=jax.ShapeDtypeStruct((NGRAPH, NCLASS), jnp.float32),
    )(pooled, post_w, post_b, ro_w, ro_b)


def kernel(x, stc_enc, dists_max, W_pre, b_pre, dc_w1, dc_b1, dc_w2, dc_b2,
           hid_w, hid_b, pos_w, pos_b, es_w, es_b, gin_w1, gin_b1, gin_w2,
           gin_b2, gcn_w, gcn_b, whp_w, whp_b, post_w, post_b, ro_w, ro_b,
           edge_index, batch, dists_argmax):
    f32 = jnp.float32
    pad = NP_ - N
    xp = jnp.pad(x, ((0, pad), (0, 0)))
    dmp = jnp.pad(dists_max, ((0, pad), (0, 0)))
    stcp = jnp.pad(stc_enc, ((0, pad), (0, 0)))
    batch2d = jnp.pad(batch.astype(jnp.int32), (0, pad),
                      constant_values=NGRAPH).reshape(GRID, 1, B)
    src = edge_index[0].astype(jnp.int32)
    dst = edge_index[1].astype(jnp.int32)
    src_p = jnp.pad(src, (0, EP + _NBUF * ECH - E))
    dst_p = jnp.pad(dst, (0, EP + _NBUF * ECH - E),
                    constant_values=NP_ - 1)
    aidx = jnp.pad(dists_argmax.reshape(-1).astype(jnp.int32),
                   (0, NAK - N * K))

    ones_ch = jnp.ones((CH, 16), f32)
    zeros16 = jnp.zeros((ROWS, 16), f32)
    zeros128 = jnp.zeros((ROWS, NHID), f32)

    acc16 = _deg_kernel(dst, ones_ch, zeros16)

    h, g, c, d = _tc1(xp, dmp, W_pre, b_pre.reshape(1, -1),
                      hid_w[:NHID], hid_w[NHID:], hid_b.reshape(1, -1),
                      dc_w1.reshape(1, NPG), dc_b1.reshape(1, NPG),
                      dc_w2.reshape(1, NPG), dc_b2.reshape(1, 1))

    sub = _gather_kernel(g, aidx)
    sub2d = sub.reshape(NP_, K * NPG)

    s = _tc2(sub2d, d, c, stcp, es_w[:SDIM], es_w[SDIM:],
             es_b.reshape(1, -1))

    hx = h
    for i in range(gin_w1.shape[0]):
        y, tcs = _tc3a(hx, s, acc16, gin_w1[i, :NHID], gin_w1[i, NHID:],
                       gcn_w[i])
        agg = _agg_kernel(y, tcs, src_p, dst_p, zeros128)
        hx, s = _tc3b(y, tcs, agg, acc16, gin_b1[i].reshape(1, -1),
                      gin_w2[i], gin_b2[i].reshape(1, -1),
                      gcn_b[i].reshape(1, -1))

    pooled = _tc4(hx, s, batch2d, whp_w[:NHID], whp_w[NHID:],
                  whp_b.reshape(1, -1))
    return _tc5(pooled, post_w, post_b.reshape(1, -1), ro_w,
                ro_b.reshape(1, -1))


# speedup vs baseline: 1.5444x; 1.5444x over previous
import functools

import jax
import jax.numpy as jnp
from jax import lax
from jax.experimental import pallas as pl
from jax.experimental.pallas import tpu as pltpu
from jax.experimental.pallas import tpu_sc as plsc

N = 10000
E = 320000
K = 32
NHID = 128
SDIM = 32
NPG = 16
NCLASS = 16
NGRAPH = 128
NP_ = 10240
NAK = NP_ * K
NC, NS = 2, 16
ROWS = NP_ // NS
CH = 80
ECH = 128
TCHUNK = 158
EPT = TCHUNK * ECH
EP = NS * EPT
EPC = E // NC
DPT = EPC // NS
DCHUNK = DPT // CH
GPT = NAK // (NC * NS)
GCH = 128
GSTAGE = 2048
_P = lax.Precision.HIGHEST
_MESH = plsc.VectorSubcoreMesh(core_axis_name="c", subcore_axis_name="s",
                               num_cores=NC, num_subcores=NS)
_SC_LINEAR = pltpu.CompilerParams(use_tc_tiling_on_sc=False)


def _dot(a, b):
    return jnp.dot(a, b, preferred_element_type=jnp.float32, precision=_P)


def _deg_body(dst_hbm, ones_hbm, zeros_hbm, out_hbm, ones_v, didx, acc_sh):
    c = lax.axis_index("c")
    s = lax.axis_index("s")
    pltpu.sync_copy(ones_hbm, ones_v)
    pltpu.sync_copy(zeros_hbm, acc_sh.at[pl.ds(s * ROWS, ROWS)])
    plsc.subcore_barrier()
    base0 = c * EPC + s * DPT

    def chunk(g, carry):
        pltpu.sync_copy(dst_hbm.at[pl.ds(base0 + g * CH, CH)], didx)
        pltpu.sync_copy(ones_v, acc_sh.at[didx], add=True)
        return carry

    lax.fori_loop(0, DCHUNK, chunk, 0)
    plsc.subcore_barrier()
    pltpu.sync_copy(acc_sh.at[pl.ds(s * ROWS, ROWS)],
                    out_hbm.at[c, pl.ds(s * ROWS, ROWS)])


_deg_kernel = pl.kernel(
    _deg_body,
    out_type=jax.ShapeDtypeStruct((NC, NP_, 16), jnp.float32),
    mesh=_MESH,
    scratch_types=[
        pltpu.VMEM((CH, 16), jnp.float32),
        pltpu.VMEM((CH,), jnp.int32),
        pltpu.VMEM_SHARED((NP_, 16), jnp.float32),
    ],
    compiler_params=_SC_LINEAR,
)


def _gather_body(g_hbm, idx_hbm, out_hbm, idx_v, stage_v, sem):
    wid = lax.axis_index("s") * NC + lax.axis_index("c")
    base0 = wid * GPT

    def outer(o, carry):
        def inner(j, carry2):
            pltpu.sync_copy(
                idx_hbm.at[pl.ds(base0 + o * GSTAGE + j * GCH, GCH)], idx_v)
            pltpu.async_copy(g_hbm.at[idx_v],
                             stage_v.at[pl.ds(j * GCH, GCH)], sem).wait()
            return carry2

        lax.fori_loop(0, GSTAGE // GCH, inner, 0)
        pltpu.sync_copy(stage_v,
                        out_hbm.at[pl.ds(base0 + o * GSTAGE, GSTAGE)])
        return carry

    lax.fori_loop(0, GPT // GSTAGE, outer, 0)


_gather_kernel = pl.kernel(
    _gather_body,
    out_type=jax.ShapeDtypeStruct((NAK, 16), jnp.float32),
    mesh=_MESH,
    scratch_types=[
        pltpu.VMEM((GCH,), jnp.int32),
        pltpu.VMEM((GSTAGE, 16), jnp.float32),
        pltpu.SemaphoreType.DMA,
    ],
    compiler_params=_SC_LINEAR,
)


def _agg_body(y_hbm, tc_hbm, src_hbm, dst_hbm, zeros_hbm, out_hbm,
              sidx0, sidx1, didx0, didx1, buf0, buf1, acc_sh,
              gs0, gs1, is0, is1):
    c = lax.axis_index("c")
    s = lax.axis_index("s")
    base0 = s * EPT

    def iload(g, sidx, didx, sem):
        pltpu.async_copy(src_hbm.at[pl.ds(base0 + g * ECH, ECH)], sidx, sem)
        pltpu.async_copy(dst_hbm.at[pl.ds(base0 + g * ECH, ECH)], didx, sem)

    def iwait(g, sidx, didx, sem):
        pltpu.make_async_copy(
            src_hbm.at[pl.ds(base0 + g * ECH, ECH)], sidx, sem).wait()
        pltpu.make_async_copy(
            dst_hbm.at[pl.ds(base0 + g * ECH, ECH)], didx, sem).wait()

    def gstart(sidx, buf, sem):
        @pl.when(c == 0)
        def _():
            pltpu.async_copy(y_hbm.at[sidx], buf, sem)

        @pl.when(c == 1)
        def _():
            pltpu.async_copy(tc_hbm.at[sidx], buf, sem)

    def gwait(sidx, buf, sem):
        pltpu.make_async_copy(y_hbm.at[sidx], buf, sem).wait()

    def scat(buf, didx):
        pltpu.sync_copy(buf, acc_sh.at[didx], add=True)

    iload(0, sidx0, didx0, is0)
    pltpu.sync_copy(zeros_hbm, acc_sh.at[pl.ds(s * ROWS, ROWS)])
    plsc.subcore_barrier()
    iwait(0, sidx0, didx0, is0)
    gstart(sidx0, buf0, gs0)
    iload(1, sidx1, didx1, is1)

    def pair(i, carry):
        iwait(2 * i + 1, sidx1, didx1, is1)
        gstart(sidx1, buf1, gs1)
        gwait(sidx0, buf0, gs0)
        scat(buf0, didx0)
        iload(2 * i + 2, sidx0, didx0, is0)
        iwait(2 * i + 2, sidx0, didx0, is0)
        gstart(sidx0, buf0, gs0)
        gwait(sidx1, buf1, gs1)
        scat(buf1, didx1)
        iload(2 * i + 3, sidx1, didx1, is1)
        return carry

    lax.fori_loop(0, TCHUNK // 2, pair, 0)
    iwait(TCHUNK + 1, sidx1, didx1, is1)
    gwait(sidx0, buf0, gs0)
    plsc.subcore_barrier()
    pltpu.sync_copy(acc_sh.at[pl.ds(s * ROWS, ROWS)],
                    out_hbm.at[c, pl.ds(s * ROWS, ROWS)])


_agg_kernel = pl.kernel(
    _agg_body,
    out_type=jax.ShapeDtypeStruct((NC, NP_, NHID), jnp.float32),
    mesh=_MESH,
    scratch_types=[
        pltpu.VMEM((ECH,), jnp.int32),
        pltpu.VMEM((ECH,), jnp.int32),
        pltpu.VMEM((ECH,), jnp.int32),
        pltpu.VMEM((ECH,), jnp.int32),
        pltpu.VMEM((ECH, NHID), jnp.float32),
        pltpu.VMEM((ECH, NHID), jnp.float32),
        pltpu.VMEM_SHARED((NP_, NHID), jnp.float32),
        pltpu.SemaphoreType.DMA,
        pltpu.SemaphoreType.DMA,
        pltpu.SemaphoreType.DMA,
        pltpu.SemaphoreType.DMA,
    ],
)


B = 1024
GRID = NP_ // B


def _tc1_body(x_ref, dm_ref, wpre_ref, bpre_ref, wa_ref, wb_ref, hidb_ref,
              dc1_ref, db1_ref, dc2_ref, db2_ref,
              h_ref, g_ref, c_ref, d_ref):
    h = _dot(x_ref[...], wpre_ref[...]) + bpre_ref[...]
    h_ref[...] = h
    g_ref[...] = _dot(h, wa_ref[...])
    c_ref[...] = _dot(h, wb_ref[...]) + hidb_ref[...]
    dm = dm_ref[...]
    d = jnp.zeros_like(dm) + db2_ref[0, 0]
    for p in range(NPG):
        d = d + jax.nn.relu(dm * dc1_ref[0, p] + db1_ref[0, p]) * dc2_ref[0, p]
    d_ref[...] = d


def _tc1(xp, dmp, w_pre, b_pre, wa, wb, hid_b, dc_w1, dc_b1, dc_w2, dc_b2):
    full = lambda shp: pl.BlockSpec(shp, lambda i: (0, 0))
    smem = lambda shp: pl.BlockSpec(shp, lambda i: (0, 0),
                                    memory_space=pltpu.SMEM)
    row = lambda w: pl.BlockSpec((B, w), lambda i: (i, 0))
    return pl.pallas_call(
        _tc1_body,
        grid=(GRID,),
        in_specs=[row(128), row(K), full((128, 128)), full((1, 128)),
                  full((128, NPG)), full((128, NPG)), full((1, NPG)),
                  smem((1, NPG)), smem((1, NPG)), smem((1, NPG)),
                  smem((1, 1))],
        out_specs=[row(128), row(NPG), row(NPG), row(K)],
        out_shape=[jax.ShapeDtypeStruct((NP_, 128), jnp.float32),
                   jax.ShapeDtypeStruct((NP_, NPG), jnp.float32),
                   jax.ShapeDtypeStruct((NP_, NPG), jnp.float32),
                   jax.ShapeDtypeStruct((NP_, K), jnp.float32)],
    )(xp, dmp, w_pre, b_pre, wa, wb, hid_b, dc_w1, dc_b1, dc_w2, dc_b2)


def _layer_post(y, tc, agg_ref, acc_ref, b1_ref, w2_ref, b2_ref, gcnb_ref):
    deg = acc_ref[0, :, 0:1] + acc_ref[1, :, 0:1] + 1.0
    dis = lax.rsqrt(deg)
    hg = _dot(jax.nn.relu(y + agg_ref[0] + b1_ref[...]),
              w2_ref[...]) + b2_ref[...]
    hx = jax.nn.relu(hg)
    s = jnp.tanh(dis * (agg_ref[1] + tc) + gcnb_ref[...])
    return hx, s


def _layer_pre(hx, s, acc_ref, w1a_ref, w1b_ref, gcnw_ref):
    y = _dot(hx, w1a_ref[...]) + _dot(s, w1b_ref[...])
    hs = _dot(s, gcnw_ref[...])
    deg = acc_ref[0, :, 0:1] + acc_ref[1, :, 0:1] + 1.0
    dis = lax.rsqrt(deg)
    return y, dis * hs


def _kinit_body(sub_ref, d_ref, c_ref, stc_ref, eswa_ref, eswb_ref, esb_ref,
                h_ref, acc_ref, w1a_ref, w1b_ref, gcnw_ref, y_ref, tc_ref):
    kk = lax.broadcasted_iota(jnp.int32, (K, K * NPG), 0)
    mm = lax.broadcasted_iota(jnp.int32, (K, K * NPG), 1)
    e32 = (kk == mm // NPG).astype(jnp.float32)
    jj = lax.broadcasted_iota(jnp.int32, (NPG, K * NPG), 0)
    m2 = lax.broadcasted_iota(jnp.int32, (NPG, K * NPG), 1)
    e16 = (jj == m2 % NPG).astype(jnp.float32)
    d_exp = _dot(d_ref[...], e32)
    c_t = _dot(c_ref[...], e16)
    msgs = jax.nn.relu(d_exp * sub_ref[...] + c_t)
    x1 = _dot(msgs, e16.T) * (1.0 / K)
    s = (_dot(stc_ref[...], eswa_ref[...])
         + _dot(x1, eswb_ref[...]) + esb_ref[...])
    y_ref[...], tc_ref[...] = _layer_pre(h_ref[...], s, acc_ref,
                                         w1a_ref, w1b_ref, gcnw_ref)


def _kinit(sub2d, dp, cp, stcp, eswa, eswb, es_b, h, acc16, w1a, w1b, gcnw):
    full = lambda shp: pl.BlockSpec(shp, lambda i: (0, 0))
    row = lambda w: pl.BlockSpec((B, w), lambda i: (i, 0))
    pair = lambda w: pl.BlockSpec((NC, B, w), lambda i: (0, i, 0))
    return pl.pallas_call(
        _kinit_body,
        grid=(GRID,),
        in_specs=[row(K * NPG), row(K), row(NPG), row(SDIM),
                  full((SDIM, 128)), full((NPG, 128)), full((1, 128)),
                  row(128), pair(16),
                  full((128, 128)), full((128, 128)), full((128, 128))],
        out_specs=[row(128), row(128)],
        out_shape=[jax.ShapeDtypeStruct((NP_, 128), jnp.float32),
                   jax.ShapeDtypeStruct((NP_, 128), jnp.float32)],
    )(sub2d, dp, cp, stcp, eswa, eswb, es_b, h, acc16, w1a, w1b, gcnw)


def _kmid_body(y_ref, tc_ref, agg_ref, acc_ref, b1_ref, w2_ref, b2_ref,
               gcnb_ref, w1a_ref, w1b_ref, gcnw_ref, yo_ref, tco_ref):
    hx, s = _layer_post(y_ref[...], tc_ref[...], agg_ref, acc_ref,
                        b1_ref, w2_ref, b2_ref, gcnb_ref)
    yo_ref[...], tco_ref[...] = _layer_pre(hx, s, acc_ref,
                                           w1a_ref, w1b_ref, gcnw_ref)


def _kmid(y, tcs, agg, acc16, b1, w2, b2, gcnb, w1a, w1b, gcnw):
    full = lambda shp: pl.BlockSpec(shp, lambda i: (0, 0))
    row = lambda w: pl.BlockSpec((B, w), lambda i: (i, 0))
    pair = lambda w: pl.BlockSpec((NC, B, w), lambda i: (0, i, 0))
    return pl.pallas_call(
        _kmid_body,
        grid=(GRID,),
        in_specs=[row(128), row(128), pair(128), pair(16), full((1, 128)),
                  full((128, 128)), full((1, 128)), full((1, 128)),
                  full((128, 128)), full((128, 128)), full((128, 128))],
        out_specs=[row(128), row(128)],
        out_shape=[jax.ShapeDtypeStruct((NP_, 128), jnp.float32),
                   jax.ShapeDtypeStruct((NP_, 128), jnp.float32)],
    )(y, tcs, agg, acc16, b1, w2, b2, gcnb, w1a, w1b, gcnw)


def _kfin_body(y_ref, tc_ref, agg_ref, acc_ref, b1_ref, w2_ref, b2_ref,
               gcnb_ref, batch_ref, wa_ref, wb_ref, whpb_ref, out_ref):
    i = pl.program_id(0)
    hx, s = _layer_post(y_ref[...], tc_ref[...], agg_ref, acc_ref,
                        b1_ref, w2_ref, b2_ref, gcnb_ref)
    hx2 = _dot(hx, wa_ref[...]) + _dot(s, wb_ref[...]) + whpb_ref[...]
    gids = lax.broadcasted_iota(jnp.int32, (NGRAPH, B), 0)
    mask = (gids == batch_ref[0]).astype(jnp.float32)
    part = _dot(mask, hx2)

    @pl.when(i == 0)
    def _():
        out_ref[...] = part

    @pl.when(i > 0)
    def _():
        out_ref[...] = out_ref[...] + part


def _kfin(y, tcs, agg, acc16, b1, w2, b2, gcnb, batch2d, wa, wb, whp_b):
    full = lambda shp: pl.BlockSpec(shp, lambda i: (0, 0))
    row = lambda w: pl.BlockSpec((B, w), lambda i: (i, 0))
    pair = lambda w: pl.BlockSpec((NC, B, w), lambda i: (0, i, 0))
    return pl.pallas_call(
        _kfin_body,
        grid=(GRID,),
        in_specs=[row(128), row(128), pair(128), pair(16), full((1, 128)),
                  full((128, 128)), full((1, 128)), full((1, 128)),
                  pl.BlockSpec((1, 1, B), lambda i: (i, 0, 0)),
                  full((128, 128)), full((128, 128)), full((1, 128))],
        out_specs=full((NGRAPH, 128)),
        out_shape=jax.ShapeDtypeStruct((NGRAPH, 128), jnp.float32),
    )(y, tcs, agg, acc16, b1, w2, b2, gcnb, batch2d, wa, wb, whp_b)


def _tc5_body(pool_ref, pw_ref, pb_ref, rw_ref, rb_ref, out_ref):
    p = jax.nn.relu(_dot(pool_ref[...], pw_ref[...]) + pb_ref[...])
    lg = _dot(p, rw_ref[...]) + rb_ref[...]
    m = jnp.max(lg, axis=1, keepdims=True)
    e = lg - m
    out_ref[...] = e - jnp.log(jnp.sum(jnp.exp(e), axis=1, keepdims=True))


def _tc5(pooled, post_w, post_b, ro_w, ro_b):
    full = lambda shp: pl.BlockSpec(shp, lambda i: (0, 0))
    return pl.pallas_call(
        _tc5_body,
        grid=(1,),
        in_specs=[full((NGRAPH, 128)), full((128, 128)), full((1, 128)),
                  full((128, NCLASS)), full((1, NCLASS))],
        out_specs=full((NGRAPH, NCLASS)),
        out_shape=jax.ShapeDtypeStruct((NGRAPH, NCLASS), jnp.float32),
    )(pooled, post_w, post_b, ro_w, ro_b)


def kernel(x, stc_enc, dists_max, W_pre, b_pre, dc_w1, dc_b1, dc_w2, dc_b2,
           hid_w, hid_b, pos_w, pos_b, es_w, es_b, gin_w1, gin_b1, gin_w2,
           gin_b2, gcn_w, gcn_b, whp_w, whp_b, post_w, post_b, ro_w, ro_b,
           edge_index, batch, dists_argmax):
    f32 = jnp.float32
    pad = NP_ - N
    xp = jnp.pad(x, ((0, pad), (0, 0)))
    dmp = jnp.pad(dists_max, ((0, pad), (0, 0)))
    stcp = jnp.pad(stc_enc, ((0, pad), (0, 0)))
    batch2d = jnp.pad(batch.astype(jnp.int32), (0, pad),
                      constant_values=NGRAPH).reshape(GRID, 1, B)
    src = edge_index[0].astype(jnp.int32)
    dst = edge_index[1].astype(jnp.int32)
    src_p = jnp.pad(src, (0, EP + 2 * ECH - E))
    dst_p = jnp.pad(dst, (0, EP + 2 * ECH - E), constant_values=NP_ - 1)
    aidx = jnp.pad(dists_argmax.reshape(-1).astype(jnp.int32),
                   (0, NAK - N * K))

    ones_ch = jnp.ones((CH, 16), f32)
    zeros16 = jnp.zeros((ROWS, 16), f32)
    zeros128 = jnp.zeros((ROWS, NHID), f32)

    acc16 = _deg_kernel(dst, ones_ch, zeros16)

    h, g, c, d = _tc1(xp, dmp, W_pre, b_pre.reshape(1, -1),
                      hid_w[:NHID], hid_w[NHID:], hid_b.reshape(1, -1),
                      dc_w1.reshape(1, NPG), dc_b1.reshape(1, NPG),
                      dc_w2.reshape(1, NPG), dc_b2.reshape(1, 1))

    sub = _gather_kernel(g, aidx)
    sub2d = sub.reshape(NP_, K * NPG)

    y, tcs = _kinit(sub2d, d, c, stcp, es_w[:SDIM], es_w[SDIM:],
                    es_b.reshape(1, -1), h, acc16,
                    gin_w1[0, :NHID], gin_w1[0, NHID:], gcn_w[0])

    nl = gin_w1.shape[0]
    for i in range(nl - 1):
        agg = _agg_kernel(y, tcs, src_p, dst_p, zeros128)
        y, tcs = _kmid(y, tcs, agg, acc16, gin_b1[i].reshape(1, -1),
                       gin_w2[i], gin_b2[i].reshape(1, -1),
                       gcn_b[i].reshape(1, -1),
                       gin_w1[i + 1, :NHID], gin_w1[i + 1, NHID:],
                       gcn_w[i + 1])

    agg = _agg_kernel(y, tcs, src_p, dst_p, zeros128)
    pooled = _kfin(y, tcs, agg, acc16, gin_b1[nl - 1].reshape(1, -1),
                   gin_w2[nl - 1], gin_b2[nl - 1].reshape(1, -1),
                   gcn_b[nl - 1].reshape(1, -1), batch2d,
                   whp_w[:NHID], whp_w[NHID:], whp_b.reshape(1, -1))
    return _tc5(pooled, post_w, post_b.reshape(1, -1), ro_w,
                ro_b.reshape(1, -1))


# speedup vs baseline: 1.5480x; 1.0023x over previous
import functools

import jax
import jax.numpy as jnp
from jax import lax
from jax.experimental import pallas as pl
from jax.experimental.pallas import tpu as pltpu
from jax.experimental.pallas import tpu_sc as plsc

N = 10000
E = 320000
K = 32
NHID = 128
SDIM = 32
NPG = 16
NCLASS = 16
NGRAPH = 128
NP_ = 10240
NAK = NP_ * K
NC, NS = 2, 16
ROWS = NP_ // NS
CH = 80
ECH = 128
TCHUNK = 158
EPT = TCHUNK * ECH
EP = NS * EPT
EPC = E // NC
DPT = EPC // NS
DCHUNK = DPT // CH
GPT = NAK // (NC * NS)
GCH = 128
GSTAGE = 2048
_P = lax.Precision.HIGHEST
_MESH = plsc.VectorSubcoreMesh(core_axis_name="c", subcore_axis_name="s",
                               num_cores=NC, num_subcores=NS)
_SC_LINEAR = pltpu.CompilerParams(use_tc_tiling_on_sc=False)


def _dot(a, b):
    return jnp.dot(a, b, preferred_element_type=jnp.float32, precision=_P)


def _deg_body(dst_hbm, ones_hbm, zeros_hbm, out_hbm, ones_v, didx, acc_sh):
    c = lax.axis_index("c")
    s = lax.axis_index("s")
    pltpu.sync_copy(ones_hbm, ones_v)
    pltpu.sync_copy(zeros_hbm, acc_sh.at[pl.ds(s * ROWS, ROWS)])
    plsc.subcore_barrier()
    base0 = c * EPC + s * DPT

    def chunk(g, carry):
        pltpu.sync_copy(dst_hbm.at[pl.ds(base0 + g * CH, CH)], didx)
        pltpu.sync_copy(ones_v, acc_sh.at[didx], add=True)
        return carry

    lax.fori_loop(0, DCHUNK, chunk, 0)
    plsc.subcore_barrier()
    pltpu.sync_copy(acc_sh.at[pl.ds(s * ROWS, ROWS)],
                    out_hbm.at[c, pl.ds(s * ROWS, ROWS)])


_deg_kernel = pl.kernel(
    _deg_body,
    out_type=jax.ShapeDtypeStruct((NC, NP_, 16), jnp.float32),
    mesh=_MESH,
    scratch_types=[
        pltpu.VMEM((CH, 16), jnp.float32),
        pltpu.VMEM((CH,), jnp.int32),
        pltpu.VMEM_SHARED((NP_, 16), jnp.float32),
    ],
    compiler_params=_SC_LINEAR,
)


_GNB = 4


def _gather_body(g_hbm, idx_hbm, out_hbm, *scr):
    idxs = scr[0:_GNB]
    bufs = scr[_GNB:2 * _GNB]
    gsem = scr[2 * _GNB:3 * _GNB]
    isem = scr[3 * _GNB:4 * _GNB]
    wsem = scr[4 * _GNB:5 * _GNB]
    wid = lax.axis_index("s") * NC + lax.axis_index("c")
    base0 = wid * GPT

    def iload(g, k):
        pltpu.async_copy(idx_hbm.at[pl.ds(base0 + g * GCH, GCH)],
                         idxs[k], isem[k])

    def iwait(g, k):
        pltpu.make_async_copy(idx_hbm.at[pl.ds(base0 + g * GCH, GCH)],
                              idxs[k], isem[k]).wait()

    def gstart(k):
        pltpu.async_copy(g_hbm.at[idxs[k]], bufs[k], gsem[k])

    def gwait(k):
        pltpu.make_async_copy(g_hbm.at[idxs[k]], bufs[k], gsem[k]).wait()

    def wstart(g, k):
        pltpu.async_copy(bufs[k], out_hbm.at[pl.ds(base0 + g * GCH, GCH)],
                         wsem[k])

    def wwait(g, k):
        pltpu.make_async_copy(bufs[k],
                              out_hbm.at[pl.ds(base0 + g * GCH, GCH)],
                              wsem[k]).wait()

    for k in range(_GNB):
        iload(k, k)
    for k in range(_GNB):
        iwait(k, k)
        gstart(k)

    def ring(i, carry):
        g = _GNB * i
        for k in range(_GNB):
            gwait(k)
            wstart(g + k, k)
            iload(g + k + _GNB, k)
            iwait(g + k + _GNB, k)
            wwait(g + k, k)
            gstart(k)
        return carry

    lax.fori_loop(0, GPT // GCH // _GNB, ring, 0)
    for k in range(_GNB):
        gwait(k)


_gather_kernel = pl.kernel(
    _gather_body,
    out_type=jax.ShapeDtypeStruct((NAK, 16), jnp.float32),
    mesh=_MESH,
    scratch_types=(
        [pltpu.VMEM((GCH,), jnp.int32)] * _GNB
        + [pltpu.VMEM((GCH, 16), jnp.float32)] * _GNB
        + [pltpu.SemaphoreType.DMA] * (3 * _GNB)
    ),
    compiler_params=_SC_LINEAR,
)


def _agg_body(y_hbm, tc_hbm, src_hbm, dst_hbm, zeros_hbm, out_hbm,
              sidx0, sidx1, didx0, didx1, buf0, buf1, acc_sh,
              gs0, gs1, is0, is1):
    c = lax.axis_index("c")
    s = lax.axis_index("s")
    base0 = s * EPT

    def iload(g, sidx, didx, sem):
        pltpu.async_copy(src_hbm.at[pl.ds(base0 + g * ECH, ECH)], sidx, sem)
        pltpu.async_copy(dst_hbm.at[pl.ds(base0 + g * ECH, ECH)], didx, sem)

    def iwait(g, sidx, didx, sem):
        pltpu.make_async_copy(
            src_hbm.at[pl.ds(base0 + g * ECH, ECH)], sidx, sem).wait()
        pltpu.make_async_copy(
            dst_hbm.at[pl.ds(base0 + g * ECH, ECH)], didx, sem).wait()

    def gstart(sidx, buf, sem):
        @pl.when(c == 0)
        def _():
            pltpu.async_copy(y_hbm.at[sidx], buf, sem)

        @pl.when(c == 1)
        def _():
            pltpu.async_copy(tc_hbm.at[sidx], buf, sem)

    def gwait(sidx, buf, sem):
        pltpu.make_async_copy(y_hbm.at[sidx], buf, sem).wait()

    def scat(buf, didx):
        pltpu.sync_copy(buf, acc_sh.at[didx], add=True)

    iload(0, sidx0, didx0, is0)
    pltpu.sync_copy(zeros_hbm, acc_sh.at[pl.ds(s * ROWS, ROWS)])
    plsc.subcore_barrier()
    iwait(0, sidx0, didx0, is0)
    gstart(sidx0, buf0, gs0)
    iload(1, sidx1, didx1, is1)

    def pair(i, carry):
        iwait(2 * i + 1, sidx1, didx1, is1)
        gstart(sidx1, buf1, gs1)
        gwait(sidx0, buf0, gs0)
        scat(buf0, didx0)
        iload(2 * i + 2, sidx0, didx0, is0)
        iwait(2 * i + 2, sidx0, didx0, is0)
        gstart(sidx0, buf0, gs0)
        gwait(sidx1, buf1, gs1)
        scat(buf1, didx1)
        iload(2 * i + 3, sidx1, didx1, is1)
        return carry

    lax.fori_loop(0, TCHUNK // 2, pair, 0)
    iwait(TCHUNK + 1, sidx1, didx1, is1)
    gwait(sidx0, buf0, gs0)
    plsc.subcore_barrier()
    pltpu.sync_copy(acc_sh.at[pl.ds(s * ROWS, ROWS)],
                    out_hbm.at[c, pl.ds(s * ROWS, ROWS)])


_agg_kernel = pl.kernel(
    _agg_body,
    out_type=jax.ShapeDtypeStruct((NC, NP_, NHID), jnp.float32),
    mesh=_MESH,
    scratch_types=[
        pltpu.VMEM((ECH,), jnp.int32),
        pltpu.VMEM((ECH,), jnp.int32),
        pltpu.VMEM((ECH,), jnp.int32),
        pltpu.VMEM((ECH,), jnp.int32),
        pltpu.VMEM((ECH, NHID), jnp.float32),
        pltpu.VMEM((ECH, NHID), jnp.float32),
        pltpu.VMEM_SHARED((NP_, NHID), jnp.float32),
        pltpu.SemaphoreType.DMA,
        pltpu.SemaphoreType.DMA,
        pltpu.SemaphoreType.DMA,
        pltpu.SemaphoreType.DMA,
    ],
)


B = 1024
GRID = NP_ // B


def _tc1_body(x_ref, dm_ref, wpre_ref, bpre_ref, wa_ref, wb_ref, hidb_ref,
              dc1_ref, db1_ref, dc2_ref, db2_ref,
              h_ref, g_ref, c_ref, d_ref):
    h = _dot(x_ref[...], wpre_ref[...]) + bpre_ref[...]
    h_ref[...] = h
    g_ref[...] = _dot(h, wa_ref[...])
    c_ref[...] = _dot(h, wb_ref[...]) + hidb_ref[...]
    dm = dm_ref[...]
    d = jnp.zeros_like(dm) + db2_ref[0, 0]
    for p in range(NPG):
        d = d + jax.nn.relu(dm * dc1_ref[0, p] + db1_ref[0, p]) * dc2_ref[0, p]
    d_ref[...] = d


def _tc1(xp, dmp, w_pre, b_pre, wa, wb, hid_b, dc_w1, dc_b1, dc_w2, dc_b2):
    full = lambda shp: pl.BlockSpec(shp, lambda i: (0, 0))
    smem = lambda shp: pl.BlockSpec(shp, lambda i: (0, 0),
                                    memory_space=pltpu.SMEM)
    row = lambda w: pl.BlockSpec((B, w), lambda i: (i, 0))
    return pl.pallas_call(
        _tc1_body,
        grid=(GRID,),
        in_specs=[row(128), row(K), full((128, 128)), full((1, 128)),
                  full((128, NPG)), full((128, NPG)), full((1, NPG)),
                  smem((1, NPG)), smem((1, NPG)), smem((1, NPG)),
                  smem((1, 1))],
        out_specs=[row(128), row(NPG), row(NPG), row(K)],
        out_shape=[jax.ShapeDtypeStruct((NP_, 128), jnp.float32),
                   jax.ShapeDtypeStruct((NP_, NPG), jnp.float32),
                   jax.ShapeDtypeStruct((NP_, NPG), jnp.float32),
                   jax.ShapeDtypeStruct((NP_, K), jnp.float32)],
    )(xp, dmp, w_pre, b_pre, wa, wb, hid_b, dc_w1, dc_b1, dc_w2, dc_b2)


def _layer_post(y, tc, agg_ref, acc_ref, b1_ref, w2_ref, b2_ref, gcnb_ref):
    deg = acc_ref[0, :, 0:1] + acc_ref[1, :, 0:1] + 1.0
    dis = lax.rsqrt(deg)
    hg = _dot(jax.nn.relu(y + agg_ref[0] + b1_ref[...]),
              w2_ref[...]) + b2_ref[...]
    hx = jax.nn.relu(hg)
    s = jnp.tanh(dis * (agg_ref[1] + tc) + gcnb_ref[...])
    return hx, s


def _layer_pre(hx, s, acc_ref, w1a_ref, w1b_ref, gcnw_ref):
    y = _dot(hx, w1a_ref[...]) + _dot(s, w1b_ref[...])
    hs = _dot(s, gcnw_ref[...])
    deg = acc_ref[0, :, 0:1] + acc_ref[1, :, 0:1] + 1.0
    dis = lax.rsqrt(deg)
    return y, dis * hs


def _kinit_body(sub_ref, d_ref, c_ref, stc_ref, eswa_ref, eswb_ref, esb_ref,
                h_ref, acc_ref, w1a_ref, w1b_ref, gcnw_ref, y_ref, tc_ref):
    kk = lax.broadcasted_iota(jnp.int32, (K, K * NPG), 0)
    mm = lax.broadcasted_iota(jnp.int32, (K, K * NPG), 1)
    e32 = (kk == mm // NPG).astype(jnp.float32)
    jj = lax.broadcasted_iota(jnp.int32, (NPG, K * NPG), 0)
    m2 = lax.broadcasted_iota(jnp.int32, (NPG, K * NPG), 1)
    e16 = (jj == m2 % NPG).astype(jnp.float32)
    d_exp = _dot(d_ref[...], e32)
    c_t = _dot(c_ref[...], e16)
    msgs = jax.nn.relu(d_exp * sub_ref[...] + c_t)
    x1 = _dot(msgs, e16.T) * (1.0 / K)
    s = (_dot(stc_ref[...], eswa_ref[...])
         + _dot(x1, eswb_ref[...]) + esb_ref[...])
    y_ref[...], tc_ref[...] = _layer_pre(h_ref[...], s, acc_ref,
                                         w1a_ref, w1b_ref, gcnw_ref)


def _kinit(sub2d, dp, cp, stcp, eswa, eswb, es_b, h, acc16, w1a, w1b, gcnw):
    full = lambda shp: pl.BlockSpec(shp, lambda i: (0, 0))
    row = lambda w: pl.BlockSpec((B, w), lambda i: (i, 0))
    pair = lambda w: pl.BlockSpec((NC, B, w), lambda i: (0, i, 0))
    return pl.pallas_call(
        _kinit_body,
        grid=(GRID,),
        in_specs=[row(K * NPG), row(K), row(NPG), row(SDIM),
                  full((SDIM, 128)), full((NPG, 128)), full((1, 128)),
                  row(128), pair(16),
                  full((128, 128)), full((128, 128)), full((128, 128))],
        out_specs=[row(128), row(128)],
        out_shape=[jax.ShapeDtypeStruct((NP_, 128), jnp.float32),
                   jax.ShapeDtypeStruct((NP_, 128), jnp.float32)],
    )(sub2d, dp, cp, stcp, eswa, eswb, es_b, h, acc16, w1a, w1b, gcnw)


def _kmid_body(y_ref, tc_ref, agg_ref, acc_ref, b1_ref, w2_ref, b2_ref,
               gcnb_ref, w1a_ref, w1b_ref, gcnw_ref, yo_ref, tco_ref):
    hx, s = _layer_post(y_ref[...], tc_ref[...], agg_ref, acc_ref,
                        b1_ref, w2_ref, b2_ref, gcnb_ref)
    yo_ref[...], tco_ref[...] = _layer_pre(hx, s, acc_ref,
                                           w1a_ref, w1b_ref, gcnw_ref)


def _kmid(y, tcs, agg, acc16, b1, w2, b2, gcnb, w1a, w1b, gcnw):
    full = lambda shp: pl.BlockSpec(shp, lambda i: (0, 0))
    row = lambda w: pl.BlockSpec((B, w), lambda i: (i, 0))
    pair = lambda w: pl.BlockSpec((NC, B, w), lambda i: (0, i, 0))
    return pl.pallas_call(
        _kmid_body,
        grid=(GRID,),
        in_specs=[row(128), row(128), pair(128), pair(16), full((1, 128)),
                  full((128, 128)), full((1, 128)), full((1, 128)),
                  full((128, 128)), full((128, 128)), full((128, 128))],
        out_specs=[row(128), row(128)],
        out_shape=[jax.ShapeDtypeStruct((NP_, 128), jnp.float32),
                   jax.ShapeDtypeStruct((NP_, 128), jnp.float32)],
    )(y, tcs, agg, acc16, b1, w2, b2, gcnb, w1a, w1b, gcnw)


def _kfin_body(y_ref, tc_ref, agg_ref, acc_ref, b1_ref, w2_ref, b2_ref,
               gcnb_ref, batch_ref, wa_ref, wb_ref, whpb_ref, out_ref):
    i = pl.program_id(0)
    hx, s = _layer_post(y_ref[...], tc_ref[...], agg_ref, acc_ref,
                        b1_ref, w2_ref, b2_ref, gcnb_ref)
    hx2 = _dot(hx, wa_ref[...]) + _dot(s, wb_ref[...]) + whpb_ref[...]
    gids = lax.broadcasted_iota(jnp.int32, (NGRAPH, B), 0)
    mask = (gids == batch_ref[0]).astype(jnp.float32)
    part = _dot(mask, hx2)

    @pl.when(i == 0)
    def _():
        out_ref[...] = part

    @pl.when(i > 0)
    def _():
        out_ref[...] = out_ref[...] + part


def _kfin(y, tcs, agg, acc16, b1, w2, b2, gcnb, batch2d, wa, wb, whp_b):
    full = lambda shp: pl.BlockSpec(shp, lambda i: (0, 0))
    row = lambda w: pl.BlockSpec((B, w), lambda i: (i, 0))
    pair = lambda w: pl.BlockSpec((NC, B, w), lambda i: (0, i, 0))
    return pl.pallas_call(
        _kfin_body,
        grid=(GRID,),
        in_specs=[row(128), row(128), pair(128), pair(16), full((1, 128)),
                  full((128, 128)), full((1, 128)), full((1, 128)),
                  pl.BlockSpec((1, 1, B), lambda i: (i, 0, 0)),
                  full((128, 128)), full((128, 128)), full((1, 128))],
        out_specs=full((NGRAPH, 128)),
        out_shape=jax.ShapeDtypeStruct((NGRAPH, 128), jnp.float32),
    )(y, tcs, agg, acc16, b1, w2, b2, gcnb, batch2d, wa, wb, whp_b)


def _tc5_body(pool_ref, pw_ref, pb_ref, rw_ref, rb_ref, out_ref):
    p = jax.nn.relu(_dot(pool_ref[...], pw_ref[...]) + pb_ref[...])
    lg = _dot(p, rw_ref[...]) + rb_ref[...]
    m = jnp.max(lg, axis=1, keepdims=True)
    e = lg - m
    out_ref[...] = e - jnp.log(jnp.sum(jnp.exp(e), axis=1, keepdims=True))


def _tc5(pooled, post_w, post_b, ro_w, ro_b):
    full = lambda shp: pl.BlockSpec(shp, lambda i: (0, 0))
    return pl.pallas_call(
        _tc5_body,
        grid=(1,),
        in_specs=[full((NGRAPH, 128)), full((128, 128)), full((1, 128)),
                  full((128, NCLASS)), full((1, NCLASS))],
        out_specs=full((NGRAPH, NCLASS)),
        out_shape=jax.ShapeDtypeStruct((NGRAPH, NCLASS), jnp.float32),
    )(pooled, post_w, post_b, ro_w, ro_b)


def kernel(x, stc_enc, dists_max, W_pre, b_pre, dc_w1, dc_b1, dc_w2, dc_b2,
           hid_w, hid_b, pos_w, pos_b, es_w, es_b, gin_w1, gin_b1, gin_w2,
           gin_b2, gcn_w, gcn_b, whp_w, whp_b, post_w, post_b, ro_w, ro_b,
           edge_index, batch, dists_argmax):
    f32 = jnp.float32
    pad = NP_ - N
    xp = jnp.pad(x, ((0, pad), (0, 0)))
    dmp = jnp.pad(dists_max, ((0, pad), (0, 0)))
    stcp = jnp.pad(stc_enc, ((0, pad), (0, 0)))
    batch2d = jnp.pad(batch.astype(jnp.int32), (0, pad),
                      constant_values=NGRAPH).reshape(GRID, 1, B)
    src = edge_index[0].astype(jnp.int32)
    dst = edge_index[1].astype(jnp.int32)
    src_p = jnp.pad(src, (0, EP + 2 * ECH - E))
    dst_p = jnp.pad(dst, (0, EP + 2 * ECH - E), constant_values=NP_ - 1)
    aidx = jnp.pad(dists_argmax.reshape(-1).astype(jnp.int32),
                   (0, NAK + _GNB * GCH - N * K))

    ones_ch = jnp.ones((CH, 16), f32)
    zeros16 = jnp.zeros((ROWS, 16), f32)
    zeros128 = jnp.zeros((ROWS, NHID), f32)

    acc16 = _deg_kernel(dst, ones_ch, zeros16)

    h, g, c, d = _tc1(xp, dmp, W_pre, b_pre.reshape(1, -1),
                      hid_w[:NHID], hid_w[NHID:], hid_b.reshape(1, -1),
                      dc_w1.reshape(1, NPG), dc_b1.reshape(1, NPG),
                      dc_w2.reshape(1, NPG), dc_b2.reshape(1, 1))

    sub = _gather_kernel(g, aidx)
    sub2d = sub.reshape(NP_, K * NPG)

    y, tcs = _kinit(sub2d, d, c, stcp, es_w[:SDIM], es_w[SDIM:],
                    es_b.reshape(1, -1), h, acc16,
                    gin_w1[0, :NHID], gin_w1[0, NHID:], gcn_w[0])

    nl = gin_w1.shape[0]
    for i in range(nl - 1):
        agg = _agg_kernel(y, tcs, src_p, dst_p, zeros128)
        y, tcs = _kmid(y, tcs, agg, acc16, gin_b1[i].reshape(1, -1),
                       gin_w2[i], gin_b2[i].reshape(1, -1),
                       gcn_b[i].reshape(1, -1),
                       gin_w1[i + 1, :NHID], gin_w1[i + 1, NHID:],
                       gcn_w[i + 1])

    agg = _agg_kernel(y, tcs, src_p, dst_p, zeros128)
    pooled = _kfin(y, tcs, agg, acc16, gin_b1[nl - 1].reshape(1, -1),
                   gin_w2[nl - 1], gin_b2[nl - 1].reshape(1, -1),
                   gcn_b[nl - 1].reshape(1, -1), batch2d,
                   whp_w[:NHID], whp_w[NHID:], whp_b.reshape(1, -1))
    return _tc5(pooled, post_w, post_b.reshape(1, -1), ro_w,
                ro_b.reshape(1, -1))


# speedup vs baseline: 1.6894x; 1.0914x over previous
import functools

import jax
import jax.numpy as jnp
from jax import lax
from jax.experimental import pallas as pl
from jax.experimental.pallas import tpu as pltpu
from jax.experimental.pallas import tpu_sc as plsc

N = 10000
E = 320000
K = 32
NHID = 128
SDIM = 32
NPG = 16
NCLASS = 16
NGRAPH = 128
NP_ = 10240
NAK = NP_ * K
NC, NS = 2, 16
ROWS = NP_ // NS
CH = 80
ECH = 120
TCHUNK = 168
EPT = TCHUNK * ECH
EP = NS * EPT
EPC = E // NC
DPT = EPC // NS
DCHUNK = DPT // CH
GPT = NAK // (NC * NS)
GCH = 128
GSTAGE = 2048
_P = lax.Precision.HIGHEST
_MESH = plsc.VectorSubcoreMesh(core_axis_name="c", subcore_axis_name="s",
                               num_cores=NC, num_subcores=NS)
_SC_LINEAR = pltpu.CompilerParams(use_tc_tiling_on_sc=False)


def _dot(a, b):
    return jnp.dot(a, b, preferred_element_type=jnp.float32, precision=_P)


def _deg_body(dst_hbm, ones_hbm, zeros_hbm, out_hbm, ones_v, didx, acc_sh):
    c = lax.axis_index("c")
    s = lax.axis_index("s")
    pltpu.sync_copy(ones_hbm, ones_v)
    pltpu.sync_copy(zeros_hbm, acc_sh.at[pl.ds(s * ROWS, ROWS)])
    plsc.subcore_barrier()
    base0 = c * EPC + s * DPT

    def chunk(g, carry):
        pltpu.sync_copy(dst_hbm.at[pl.ds(base0 + g * CH, CH)], didx)
        pltpu.sync_copy(ones_v, acc_sh.at[didx], add=True)
        return carry

    lax.fori_loop(0, DCHUNK, chunk, 0)
    plsc.subcore_barrier()
    pltpu.sync_copy(acc_sh.at[pl.ds(s * ROWS, ROWS)],
                    out_hbm.at[c, pl.ds(s * ROWS, ROWS)])


_deg_kernel = pl.kernel(
    _deg_body,
    out_type=jax.ShapeDtypeStruct((NC, NP_, 16), jnp.float32),
    mesh=_MESH,
    scratch_types=[
        pltpu.VMEM((CH, 16), jnp.float32),
        pltpu.VMEM((CH,), jnp.int32),
        pltpu.VMEM_SHARED((NP_, 16), jnp.float32),
    ],
    compiler_params=_SC_LINEAR,
)


_GNB = 4


def _gather_body(g_hbm, idx_hbm, out_hbm, *scr):
    idxs = scr[0:_GNB]
    bufs = scr[_GNB:2 * _GNB]
    gsem = scr[2 * _GNB:3 * _GNB]
    isem = scr[3 * _GNB:4 * _GNB]
    wsem = scr[4 * _GNB:5 * _GNB]
    wid = lax.axis_index("s") * NC + lax.axis_index("c")
    base0 = wid * GPT

    def iload(g, k):
        pltpu.async_copy(idx_hbm.at[pl.ds(base0 + g * GCH, GCH)],
                         idxs[k], isem[k])

    def iwait(g, k):
        pltpu.make_async_copy(idx_hbm.at[pl.ds(base0 + g * GCH, GCH)],
                              idxs[k], isem[k]).wait()

    def gstart(k):
        pltpu.async_copy(g_hbm.at[idxs[k]], bufs[k], gsem[k])

    def gwait(k):
        pltpu.make_async_copy(g_hbm.at[idxs[k]], bufs[k], gsem[k]).wait()

    def wstart(g, k):
        pltpu.async_copy(bufs[k], out_hbm.at[pl.ds(base0 + g * GCH, GCH)],
                         wsem[k])

    def wwait(g, k):
        pltpu.make_async_copy(bufs[k],
                              out_hbm.at[pl.ds(base0 + g * GCH, GCH)],
                              wsem[k]).wait()

    for k in range(_GNB):
        iload(k, k)
    for k in range(_GNB):
        iwait(k, k)
        gstart(k)

    def ring(i, carry):
        g = _GNB * i
        for k in range(_GNB):
            gwait(k)
            wstart(g + k, k)
            iload(g + k + _GNB, k)
            iwait(g + k + _GNB, k)
            wwait(g + k, k)
            gstart(k)
        return carry

    lax.fori_loop(0, GPT // GCH // _GNB, ring, 0)
    for k in range(_GNB):
        gwait(k)


_gather_kernel = pl.kernel(
    _gather_body,
    out_type=jax.ShapeDtypeStruct((NAK, 16), jnp.float32),
    mesh=_MESH,
    scratch_types=(
        [pltpu.VMEM((GCH,), jnp.int32)] * _GNB
        + [pltpu.VMEM((GCH, 16), jnp.float32)] * _GNB
        + [pltpu.SemaphoreType.DMA] * (3 * _GNB)
    ),
    compiler_params=_SC_LINEAR,
)


_ANB = 3


def _agg_body(y_hbm, tc_hbm, src_hbm, dst_hbm, zeros_hbm, out_hbm, *scr):
    sidx = scr[0:_ANB]
    didx = scr[_ANB:2 * _ANB]
    bufs = scr[2 * _ANB:3 * _ANB]
    acc_sh = scr[3 * _ANB]
    gsem = scr[3 * _ANB + 1:4 * _ANB + 1]
    isem = scr[4 * _ANB + 1:5 * _ANB + 1]
    c = lax.axis_index("c")
    s = lax.axis_index("s")
    base0 = s * EPT

    def iload(g, k):
        pltpu.async_copy(src_hbm.at[pl.ds(base0 + g * ECH, ECH)],
                         sidx[k], isem[k])
        pltpu.async_copy(dst_hbm.at[pl.ds(base0 + g * ECH, ECH)],
                         didx[k], isem[k])

    def iwait(g, k):
        pltpu.make_async_copy(
            src_hbm.at[pl.ds(base0 + g * ECH, ECH)], sidx[k], isem[k]).wait()
        pltpu.make_async_copy(
            dst_hbm.at[pl.ds(base0 + g * ECH, ECH)], didx[k], isem[k]).wait()

    def gstart(k):
        @pl.when(c == 0)
        def _():
            pltpu.async_copy(y_hbm.at[sidx[k]], bufs[k], gsem[k])

        @pl.when(c == 1)
        def _():
            pltpu.async_copy(tc_hbm.at[sidx[k]], bufs[k], gsem[k])

    def gwait(k):
        pltpu.make_async_copy(y_hbm.at[sidx[k]], bufs[k], gsem[k]).wait()

    def scat(k):
        pltpu.sync_copy(bufs[k], acc_sh.at[didx[k]], add=True)

    for k in range(_ANB):
        iload(k, k)
    pltpu.sync_copy(zeros_hbm, acc_sh.at[pl.ds(s * ROWS, ROWS)])
    plsc.subcore_barrier()
    for k in range(_ANB):
        iwait(k, k)
        gstart(k)

    def ring(i, carry):
        g = _ANB * i
        for k in range(_ANB):
            gwait(k)
            scat(k)
            iload(g + k + _ANB, k)
            iwait(g + k + _ANB, k)
            gstart(k)
        return carry

    lax.fori_loop(0, TCHUNK // _ANB, ring, 0)
    for k in range(_ANB):
        gwait(k)
    plsc.subcore_barrier()
    pltpu.sync_copy(acc_sh.at[pl.ds(s * ROWS, ROWS)],
                    out_hbm.at[c, pl.ds(s * ROWS, ROWS)])


_agg_kernel = pl.kernel(
    _agg_body,
    out_type=jax.ShapeDtypeStruct((NC, NP_, NHID), jnp.float32),
    mesh=_MESH,
    scratch_types=(
        [pltpu.VMEM((ECH,), jnp.int32)] * (2 * _ANB)
        + [pltpu.VMEM((ECH, NHID), jnp.float32)] * _ANB
        + [pltpu.VMEM_SHARED((NP_, NHID), jnp.float32)]
        + [pltpu.SemaphoreType.DMA] * (2 * _ANB)
    ),
)


B = 1024
GRID = NP_ // B


def _tc1_body(x_ref, dm_ref, wpre_ref, bpre_ref, wa_ref, wb_ref, hidb_ref,
              dc1_ref, db1_ref, dc2_ref, db2_ref,
              h_ref, g_ref, c_ref, d_ref):
    h = _dot(x_ref[...], wpre_ref[...]) + bpre_ref[...]
    h_ref[...] = h
    g_ref[...] = _dot(h, wa_ref[...])
    c_ref[...] = _dot(h, wb_ref[...]) + hidb_ref[...]
    dm = dm_ref[...]
    d = jnp.zeros_like(dm) + db2_ref[0, 0]
    for p in range(NPG):
        d = d + jax.nn.relu(dm * dc1_ref[0, p] + db1_ref[0, p]) * dc2_ref[0, p]
    d_ref[...] = d


def _tc1(xp, dmp, w_pre, b_pre, wa, wb, hid_b, dc_w1, dc_b1, dc_w2, dc_b2):
    full = lambda shp: pl.BlockSpec(shp, lambda i: (0, 0))
    smem = lambda shp: pl.BlockSpec(shp, lambda i: (0, 0),
                                    memory_space=pltpu.SMEM)
    row = lambda w: pl.BlockSpec((B, w), lambda i: (i, 0))
    return pl.pallas_call(
        _tc1_body,
        grid=(GRID,),
        in_specs=[row(128), row(K), full((128, 128)), full((1, 128)),
                  full((128, NPG)), full((128, NPG)), full((1, NPG)),
                  smem((1, NPG)), smem((1, NPG)), smem((1, NPG)),
                  smem((1, 1))],
        out_specs=[row(128), row(NPG), row(NPG), row(K)],
        out_shape=[jax.ShapeDtypeStruct((NP_, 128), jnp.float32),
                   jax.ShapeDtypeStruct((NP_, NPG), jnp.float32),
                   jax.ShapeDtypeStruct((NP_, NPG), jnp.float32),
                   jax.ShapeDtypeStruct((NP_, K), jnp.float32)],
    )(xp, dmp, w_pre, b_pre, wa, wb, hid_b, dc_w1, dc_b1, dc_w2, dc_b2)


def _layer_post(y, tc, agg_ref, acc_ref, b1_ref, w2_ref, b2_ref, gcnb_ref):
    deg = acc_ref[0, :, 0:1] + acc_ref[1, :, 0:1] + 1.0
    dis = lax.rsqrt(deg)
    hg = _dot(jax.nn.relu(y + agg_ref[0] + b1_ref[...]),
              w2_ref[...]) + b2_ref[...]
    hx = jax.nn.relu(hg)
    s = jnp.tanh(dis * (agg_ref[1] + tc) + gcnb_ref[...])
    return hx, s


def _layer_pre(hx, s, acc_ref, w1a_ref, w1b_ref, gcnw_ref):
    y = _dot(hx, w1a_ref[...]) + _dot(s, w1b_ref[...])
    hs = _dot(s, gcnw_ref[...])
    deg = acc_ref[0, :, 0:1] + acc_ref[1, :, 0:1] + 1.0
    dis = lax.rsqrt(deg)
    return y, dis * hs


def _kinit_body(sub_ref, d_ref, c_ref, stc_ref, eswa_ref, eswb_ref, esb_ref,
                h_ref, acc_ref, w1a_ref, w1b_ref, gcnw_ref, y_ref, tc_ref):
    kk = lax.broadcasted_iota(jnp.int32, (K, K * NPG), 0)
    mm = lax.broadcasted_iota(jnp.int32, (K, K * NPG), 1)
    e32 = (kk == mm // NPG).astype(jnp.float32)
    jj = lax.broadcasted_iota(jnp.int32, (NPG, K * NPG), 0)
    m2 = lax.broadcasted_iota(jnp.int32, (NPG, K * NPG), 1)
    e16 = (jj == m2 % NPG).astype(jnp.float32)
    d_exp = _dot(d_ref[...], e32)
    c_t = _dot(c_ref[...], e16)
    msgs = jax.nn.relu(d_exp * sub_ref[...] + c_t)
    x1 = _dot(msgs, e16.T) * (1.0 / K)
    s = (_dot(stc_ref[...], eswa_ref[...])
         + _dot(x1, eswb_ref[...]) + esb_ref[...])
    y_ref[...], tc_ref[...] = _layer_pre(h_ref[...], s, acc_ref,
                                         w1a_ref, w1b_ref, gcnw_ref)


def _kinit(sub2d, dp, cp, stcp, eswa, eswb, es_b, h, acc16, w1a, w1b, gcnw):
    full = lambda shp: pl.BlockSpec(shp, lambda i: (0, 0))
    row = lambda w: pl.BlockSpec((B, w), lambda i: (i, 0))
    pair = lambda w: pl.BlockSpec((NC, B, w), lambda i: (0, i, 0))
    return pl.pallas_call(
        _kinit_body,
        grid=(GRID,),
        in_specs=[row(K * NPG), row(K), row(NPG), row(SDIM),
                  full((SDIM, 128)), full((NPG, 128)), full((1, 128)),
                  row(128), pair(16),
                  full((128, 128)), full((128, 128)), full((128, 128))],
        out_specs=[row(128), row(128)],
        out_shape=[jax.ShapeDtypeStruct((NP_, 128), jnp.float32),
                   jax.ShapeDtypeStruct((NP_, 128), jnp.float32)],
    )(sub2d, dp, cp, stcp, eswa, eswb, es_b, h, acc16, w1a, w1b, gcnw)


def _kmid_body(y_ref, tc_ref, agg_ref, acc_ref, b1_ref, w2_ref, b2_ref,
               gcnb_ref, w1a_ref, w1b_ref, gcnw_ref, yo_ref, tco_ref):
    hx, s = _layer_post(y_ref[...], tc_ref[...], agg_ref, acc_ref,
                        b1_ref, w2_ref, b2_ref, gcnb_ref)
    yo_ref[...], tco_ref[...] = _layer_pre(hx, s, acc_ref,
                                           w1a_ref, w1b_ref, gcnw_ref)


def _kmid(y, tcs, agg, acc16, b1, w2, b2, gcnb, w1a, w1b, gcnw):
    full = lambda shp: pl.BlockSpec(shp, lambda i: (0, 0))
    row = lambda w: pl.BlockSpec((B, w), lambda i: (i, 0))
    pair = lambda w: pl.BlockSpec((NC, B, w), lambda i: (0, i, 0))
    return pl.pallas_call(
        _kmid_body,
        grid=(GRID,),
        in_specs=[row(128), row(128), pair(128), pair(16), full((1, 128)),
                  full((128, 128)), full((1, 128)), full((1, 128)),
                  full((128, 128)), full((128, 128)), full((128, 128))],
        out_specs=[row(128), row(128)],
        out_shape=[jax.ShapeDtypeStruct((NP_, 128), jnp.float32),
                   jax.ShapeDtypeStruct((NP_, 128), jnp.float32)],
    )(y, tcs, agg, acc16, b1, w2, b2, gcnb, w1a, w1b, gcnw)


def _kfin_body(y_ref, tc_ref, agg_ref, acc_ref, b1_ref, w2_ref, b2_ref,
               gcnb_ref, batch_ref, wa_ref, wb_ref, whpb_ref, out_ref):
    i = pl.program_id(0)
    hx, s = _layer_post(y_ref[...], tc_ref[...], agg_ref, acc_ref,
                        b1_ref, w2_ref, b2_ref, gcnb_ref)
    hx2 = _dot(hx, wa_ref[...]) + _dot(s, wb_ref[...]) + whpb_ref[...]
    gids = lax.broadcasted_iota(jnp.int32, (NGRAPH, B), 0)
    mask = (gids == batch_ref[0]).astype(jnp.float32)
    part = _dot(mask, hx2)

    @pl.when(i == 0)
    def _():
        out_ref[...] = part

    @pl.when(i > 0)
    def _():
        out_ref[...] = out_ref[...] + part


def _kfin(y, tcs, agg, acc16, b1, w2, b2, gcnb, batch2d, wa, wb, whp_b):
    full = lambda shp: pl.BlockSpec(shp, lambda i: (0, 0))
    row = lambda w: pl.BlockSpec((B, w), lambda i: (i, 0))
    pair = lambda w: pl.BlockSpec((NC, B, w), lambda i: (0, i, 0))
    return pl.pallas_call(
        _kfin_body,
        grid=(GRID,),
        in_specs=[row(128), row(128), pair(128), pair(16), full((1, 128)),
                  full((128, 128)), full((1, 128)), full((1, 128)),
                  pl.BlockSpec((1, 1, B), lambda i: (i, 0, 0)),
                  full((128, 128)), full((128, 128)), full((1, 128))],
        out_specs=full((NGRAPH, 128)),
        out_shape=jax.ShapeDtypeStruct((NGRAPH, 128), jnp.float32),
    )(y, tcs, agg, acc16, b1, w2, b2, gcnb, batch2d, wa, wb, whp_b)


def _tc5_body(pool_ref, pw_ref, pb_ref, rw_ref, rb_ref, out_ref):
    p = jax.nn.relu(_dot(pool_ref[...], pw_ref[...]) + pb_ref[...])
    lg = _dot(p, rw_ref[...]) + rb_ref[...]
    m = jnp.max(lg, axis=1, keepdims=True)
    e = lg - m
    out_ref[...] = e - jnp.log(jnp.sum(jnp.exp(e), axis=1, keepdims=True))


def _tc5(pooled, post_w, post_b, ro_w, ro_b):
    full = lambda shp: pl.BlockSpec(shp, lambda i: (0, 0))
    return pl.pallas_call(
        _tc5_body,
        grid=(1,),
        in_specs=[full((NGRAPH, 128)), full((128, 128)), full((1, 128)),
                  full((128, NCLASS)), full((1, NCLASS))],
        out_specs=full((NGRAPH, NCLASS)),
        out_shape=jax.ShapeDtypeStruct((NGRAPH, NCLASS), jnp.float32),
    )(pooled, post_w, post_b, ro_w, ro_b)


def kernel(x, stc_enc, dists_max, W_pre, b_pre, dc_w1, dc_b1, dc_w2, dc_b2,
           hid_w, hid_b, pos_w, pos_b, es_w, es_b, gin_w1, gin_b1, gin_w2,
           gin_b2, gcn_w, gcn_b, whp_w, whp_b, post_w, post_b, ro_w, ro_b,
           edge_index, batch, dists_argmax):
    f32 = jnp.float32
    pad = NP_ - N
    xp = jnp.pad(x, ((0, pad), (0, 0)))
    dmp = jnp.pad(dists_max, ((0, pad), (0, 0)))
    stcp = jnp.pad(stc_enc, ((0, pad), (0, 0)))
    batch2d = jnp.pad(batch.astype(jnp.int32), (0, pad),
                      constant_values=NGRAPH).reshape(GRID, 1, B)
    src = edge_index[0].astype(jnp.int32)
    dst = edge_index[1].astype(jnp.int32)
    src_p = jnp.pad(src, (0, EP + _ANB * ECH - E))
    dst_p = jnp.pad(dst, (0, EP + _ANB * ECH - E), constant_values=NP_ - 1)
    aidx = jnp.pad(dists_argmax.reshape(-1).astype(jnp.int32),
                   (0, NAK + _GNB * GCH - N * K))

    ones_ch = jnp.ones((CH, 16), f32)
    zeros16 = jnp.zeros((ROWS, 16), f32)
    zeros128 = jnp.zeros((ROWS, NHID), f32)

    acc16 = _deg_kernel(dst, ones_ch, zeros16)

    h, g, c, d = _tc1(xp, dmp, W_pre, b_pre.reshape(1, -1),
                      hid_w[:NHID], hid_w[NHID:], hid_b.reshape(1, -1),
                      dc_w1.reshape(1, NPG), dc_b1.reshape(1, NPG),
                      dc_w2.reshape(1, NPG), dc_b2.reshape(1, 1))

    sub = _gather_kernel(g, aidx)
    sub2d = sub.reshape(NP_, K * NPG)

    y, tcs = _kinit(sub2d, d, c, stcp, es_w[:SDIM], es_w[SDIM:],
                    es_b.reshape(1, -1), h, acc16,
                    gin_w1[0, :NHID], gin_w1[0, NHID:], gcn_w[0])

    nl = gin_w1.shape[0]
    for i in range(nl - 1):
        agg = _agg_kernel(y, tcs, src_p, dst_p, zeros128)
        y, tcs = _kmid(y, tcs, agg, acc16, gin_b1[i].reshape(1, -1),
                       gin_w2[i], gin_b2[i].reshape(1, -1),
                       gcn_b[i].reshape(1, -1),
                       gin_w1[i + 1, :NHID], gin_w1[i + 1, NHID:],
                       gcn_w[i + 1])

    agg = _agg_kernel(y, tcs, src_p, dst_p, zeros128)
    pooled = _kfin(y, tcs, agg, acc16, gin_b1[nl - 1].reshape(1, -1),
                   gin_w2[nl - 1], gin_b2[nl - 1].reshape(1, -1),
                   gcn_b[nl - 1].reshape(1, -1), batch2d,
                   whp_w[:NHID], whp_w[NHID:], whp_b.reshape(1, -1))
    return _tc5(pooled, post_w, post_b.reshape(1, -1), ro_w,
                ro_b.reshape(1, -1))


# speedup vs baseline: 1.7746x; 1.0504x over previous
import functools

import jax
import jax.numpy as jnp
from jax import lax
from jax.experimental import pallas as pl
from jax.experimental.pallas import tpu as pltpu
from jax.experimental.pallas import tpu_sc as plsc

N = 10000
E = 320000
K = 32
NHID = 128
SDIM = 32
NPG = 16
NCLASS = 16
NGRAPH = 128
NP_ = 10240
NAK = NP_ * K
NC, NS = 2, 16
ROWS = NP_ // NS
ECH = 120
TCHUNK = 168
EPT = TCHUNK * ECH
EP = NS * EPT
DPT = EP // NC // NS
DCHUNK = DPT // ECH
GPT = NAK // (NC * NS)
GCH = 128
GSTAGE = 2048
_P = lax.Precision.HIGHEST
_MESH = plsc.VectorSubcoreMesh(core_axis_name="c", subcore_axis_name="s",
                               num_cores=NC, num_subcores=NS)
_SC_LINEAR = pltpu.CompilerParams(use_tc_tiling_on_sc=False)


def _dot(a, b):
    return jnp.dot(a, b, preferred_element_type=jnp.float32, precision=_P)


_DNB = 4


def _deg_body(dst_hbm, ones_hbm, zeros_hbm, out_hbm, *scr):
    ones_v = scr[0]
    didx = scr[1:1 + _DNB]
    isem = scr[1 + _DNB:1 + 2 * _DNB]
    acc_sh = scr[1 + 2 * _DNB]
    c = lax.axis_index("c")
    s = lax.axis_index("s")
    base0 = c * (EP // NC) + s * DPT

    def iload(g, k):
        pltpu.async_copy(dst_hbm.at[pl.ds(base0 + g * ECH, ECH)],
                         didx[k], isem[k])

    def iwait(g, k):
        pltpu.make_async_copy(dst_hbm.at[pl.ds(base0 + g * ECH, ECH)],
                              didx[k], isem[k]).wait()

    pltpu.sync_copy(ones_hbm, ones_v)
    for k in range(_DNB):
        iload(k, k)
    pltpu.sync_copy(zeros_hbm, acc_sh.at[pl.ds(s * ROWS, ROWS)])
    plsc.subcore_barrier()

    def ring(i, carry):
        g = _DNB * i
        for k in range(_DNB):
            iwait(g + k, k)
            pltpu.sync_copy(ones_v, acc_sh.at[didx[k]], add=True)
            iload(g + k + _DNB, k)
        return carry

    lax.fori_loop(0, DCHUNK // _DNB, ring, 0)
    for k in range(_DNB):
        iwait(DCHUNK + k, k)
    plsc.subcore_barrier()
    pltpu.sync_copy(acc_sh.at[pl.ds(s * ROWS, ROWS)],
                    out_hbm.at[c, pl.ds(s * ROWS, ROWS)])


_deg_kernel = pl.kernel(
    _deg_body,
    out_type=jax.ShapeDtypeStruct((NC, NP_, 16), jnp.float32),
    mesh=_MESH,
    scratch_types=(
        [pltpu.VMEM((ECH, 16), jnp.float32)]
        + [pltpu.VMEM((ECH,), jnp.int32)] * _DNB
        + [pltpu.SemaphoreType.DMA] * _DNB
        + [pltpu.VMEM_SHARED((NP_, 16), jnp.float32)]
    ),
    compiler_params=_SC_LINEAR,
)


_GNB = 4


def _gather_body(g_hbm, idx_hbm, out_hbm, *scr):
    idxs = scr[0:_GNB]
    bufs = scr[_GNB:2 * _GNB]
    gsem = scr[2 * _GNB:3 * _GNB]
    isem = scr[3 * _GNB:4 * _GNB]
    wsem = scr[4 * _GNB:5 * _GNB]
    wid = lax.axis_index("s") * NC + lax.axis_index("c")
    base0 = wid * GPT

    def iload(g, k):
        pltpu.async_copy(idx_hbm.at[pl.ds(base0 + g * GCH, GCH)],
                         idxs[k], isem[k])

    def iwait(g, k):
        pltpu.make_async_copy(idx_hbm.at[pl.ds(base0 + g * GCH, GCH)],
                              idxs[k], isem[k]).wait()

    def gstart(k):
        pltpu.async_copy(g_hbm.at[idxs[k]], bufs[k], gsem[k])

    def gwait(k):
        pltpu.make_async_copy(g_hbm.at[idxs[k]], bufs[k], gsem[k]).wait()

    def wstart(g, k):
        pltpu.async_copy(bufs[k], out_hbm.at[pl.ds(base0 + g * GCH, GCH)],
                         wsem[k])

    def wwait(g, k):
        pltpu.make_async_copy(bufs[k],
                              out_hbm.at[pl.ds(base0 + g * GCH, GCH)],
                              wsem[k]).wait()

    for k in range(_GNB):
        iload(k, k)
    for k in range(_GNB):
        iwait(k, k)
        gstart(k)

    def ring(i, carry):
        g = _GNB * i
        for k in range(_GNB):
            gwait(k)
            wstart(g + k, k)
            iload(g + k + _GNB, k)
            iwait(g + k + _GNB, k)
            wwait(g + k, k)
            gstart(k)
        return carry

    lax.fori_loop(0, GPT // GCH // _GNB, ring, 0)
    for k in range(_GNB):
        gwait(k)


_gather_kernel = pl.kernel(
    _gather_body,
    out_type=jax.ShapeDtypeStruct((NAK, 16), jnp.float32),
    mesh=_MESH,
    scratch_types=(
        [pltpu.VMEM((GCH,), jnp.int32)] * _GNB
        + [pltpu.VMEM((GCH, 16), jnp.float32)] * _GNB
        + [pltpu.SemaphoreType.DMA] * (3 * _GNB)
    ),
    compiler_params=_SC_LINEAR,
)


_ANB = 3


def _agg_body(y_hbm, tc_hbm, src_hbm, dst_hbm, zeros_hbm, out_hbm, *scr):
    sidx = scr[0:_ANB]
    didx = scr[_ANB:2 * _ANB]
    bufs = scr[2 * _ANB:3 * _ANB]
    acc_sh = scr[3 * _ANB]
    gsem = scr[3 * _ANB + 1:4 * _ANB + 1]
    isem = scr[4 * _ANB + 1:5 * _ANB + 1]
    c = lax.axis_index("c")
    s = lax.axis_index("s")
    base0 = s * EPT

    def iload(g, k):
        pltpu.async_copy(src_hbm.at[pl.ds(base0 + g * ECH, ECH)],
                         sidx[k], isem[k])
        pltpu.async_copy(dst_hbm.at[pl.ds(base0 + g * ECH, ECH)],
                         didx[k], isem[k])

    def iwait(g, k):
        pltpu.make_async_copy(
            src_hbm.at[pl.ds(base0 + g * ECH, ECH)], sidx[k], isem[k]).wait()
        pltpu.make_async_copy(
            dst_hbm.at[pl.ds(base0 + g * ECH, ECH)], didx[k], isem[k]).wait()

    def gstart(k):
        @pl.when(c == 0)
        def _():
            pltpu.async_copy(y_hbm.at[sidx[k]], bufs[k], gsem[k])

        @pl.when(c == 1)
        def _():
            pltpu.async_copy(tc_hbm.at[sidx[k]], bufs[k], gsem[k])

    def gwait(k):
        pltpu.make_async_copy(y_hbm.at[sidx[k]], bufs[k], gsem[k]).wait()

    def scat(k):
        pltpu.sync_copy(bufs[k], acc_sh.at[didx[k]], add=True)

    for k in range(_ANB):
        iload(k, k)
    pltpu.sync_copy(zeros_hbm, acc_sh.at[pl.ds(s * ROWS, ROWS)])
    plsc.subcore_barrier()
    for k in range(_ANB):
        iwait(k, k)
        gstart(k)

    def ring(i, carry):
        g = _ANB * i
        for k in range(_ANB):
            gwait(k)
            scat(k)
            iload(g + k + _ANB, k)
            iwait(g + k + _ANB, k)
            gstart(k)
        return carry

    lax.fori_loop(0, TCHUNK // _ANB, ring, 0)
    for k in range(_ANB):
        gwait(k)
    plsc.subcore_barrier()
    pltpu.sync_copy(acc_sh.at[pl.ds(s * ROWS, ROWS)],
                    out_hbm.at[c, pl.ds(s * ROWS, ROWS)])


_agg_kernel = pl.kernel(
    _agg_body,
    out_type=jax.ShapeDtypeStruct((NC, NP_, NHID), jnp.float32),
    mesh=_MESH,
    scratch_types=(
        [pltpu.VMEM((ECH,), jnp.int32)] * (2 * _ANB)
        + [pltpu.VMEM((ECH, NHID), jnp.float32)] * _ANB
        + [pltpu.VMEM_SHARED((NP_, NHID), jnp.float32)]
        + [pltpu.SemaphoreType.DMA] * (2 * _ANB)
    ),
)


B = 1024
GRID = NP_ // B


def _tc1_body(x_ref, dm_ref, wpre_ref, bpre_ref, wa_ref, wb_ref, hidb_ref,
              dc1_ref, db1_ref, dc2_ref, db2_ref,
              h_ref, g_ref, c_ref, d_ref):
    h = _dot(x_ref[...], wpre_ref[...]) + bpre_ref[...]
    h_ref[...] = h
    g_ref[...] = _dot(h, wa_ref[...])
    c_ref[...] = _dot(h, wb_ref[...]) + hidb_ref[...]
    dm = dm_ref[...]
    d = jnp.zeros_like(dm) + db2_ref[0, 0]
    for p in range(NPG):
        d = d + jax.nn.relu(dm * dc1_ref[0, p] + db1_ref[0, p]) * dc2_ref[0, p]
    d_ref[...] = d


def _tc1(xp, dmp, w_pre, b_pre, wa, wb, hid_b, dc_w1, dc_b1, dc_w2, dc_b2):
    full = lambda shp: pl.BlockSpec(shp, lambda i: (0, 0))
    smem = lambda shp: pl.BlockSpec(shp, lambda i: (0, 0),
                                    memory_space=pltpu.SMEM)
    row = lambda w: pl.BlockSpec((B, w), lambda i: (i, 0))
    return pl.pallas_call(
        _tc1_body,
        grid=(GRID,),
        in_specs=[row(128), row(K), full((128, 128)), full((1, 128)),
                  full((128, NPG)), full((128, NPG)), full((1, NPG)),
                  smem((1, NPG)), smem((1, NPG)), smem((1, NPG)),
                  smem((1, 1))],
        out_specs=[row(128), row(NPG), row(NPG), row(K)],
        out_shape=[jax.ShapeDtypeStruct((NP_, 128), jnp.float32),
                   jax.ShapeDtypeStruct((NP_, NPG), jnp.float32),
                   jax.ShapeDtypeStruct((NP_, NPG), jnp.float32),
                   jax.ShapeDtypeStruct((NP_, K), jnp.float32)],
    )(xp, dmp, w_pre, b_pre, wa, wb, hid_b, dc_w1, dc_b1, dc_w2, dc_b2)


def _layer_post(y, tc, agg_ref, acc_ref, b1_ref, w2_ref, b2_ref, gcnb_ref):
    deg = acc_ref[0, :, 0:1] + acc_ref[1, :, 0:1] + 1.0
    dis = lax.rsqrt(deg)
    hg = _dot(jax.nn.relu(y + agg_ref[0] + b1_ref[...]),
              w2_ref[...]) + b2_ref[...]
    hx = jax.nn.relu(hg)
    s = jnp.tanh(dis * (agg_ref[1] + tc) + gcnb_ref[...])
    return hx, s


def _layer_pre(hx, s, acc_ref, w1a_ref, w1b_ref, gcnw_ref):
    y = _dot(hx, w1a_ref[...]) + _dot(s, w1b_ref[...])
    hs = _dot(s, gcnw_ref[...])
    deg = acc_ref[0, :, 0:1] + acc_ref[1, :, 0:1] + 1.0
    dis = lax.rsqrt(deg)
    return y, dis * hs


def _kinit_body(sub_ref, d_ref, c_ref, stc_ref, eswa_ref, eswb_ref, esb_ref,
                h_ref, acc_ref, w1a_ref, w1b_ref, gcnw_ref, y_ref, tc_ref):
    kk = lax.broadcasted_iota(jnp.int32, (K, K * NPG), 0)
    mm = lax.broadcasted_iota(jnp.int32, (K, K * NPG), 1)
    e32 = (kk == mm // NPG).astype(jnp.float32)
    jj = lax.broadcasted_iota(jnp.int32, (NPG, K * NPG), 0)
    m2 = lax.broadcasted_iota(jnp.int32, (NPG, K * NPG), 1)
    e16 = (jj == m2 % NPG).astype(jnp.float32)
    d_exp = _dot(d_ref[...], e32)
    c_t = _dot(c_ref[...], e16)
    msgs = jax.nn.relu(d_exp * sub_ref[...] + c_t)
    x1 = _dot(msgs, e16.T) * (1.0 / K)
    s = (_dot(stc_ref[...], eswa_ref[...])
         + _dot(x1, eswb_ref[...]) + esb_ref[...])
    y_ref[...], tc_ref[...] = _layer_pre(h_ref[...], s, acc_ref,
                                         w1a_ref, w1b_ref, gcnw_ref)


def _kinit(sub2d, dp, cp, stcp, eswa, eswb, es_b, h, acc16, w1a, w1b, gcnw):
    full = lambda shp: pl.BlockSpec(shp, lambda i: (0, 0))
    row = lambda w: pl.BlockSpec((B, w), lambda i: (i, 0))
    pair = lambda w: pl.BlockSpec((NC, B, w), lambda i: (0, i, 0))
    return pl.pallas_call(
        _kinit_body,
        grid=(GRID,),
        in_specs=[row(K * NPG), row(K), row(NPG), row(SDIM),
                  full((SDIM, 128)), full((NPG, 128)), full((1, 128)),
                  row(128), pair(16),
                  full((128, 128)), full((128, 128)), full((128, 128))],
        out_specs=[row(128), row(128)],
        out_shape=[jax.ShapeDtypeStruct((NP_, 128), jnp.float32),
                   jax.ShapeDtypeStruct((NP_, 128), jnp.float32)],
    )(sub2d, dp, cp, stcp, eswa, eswb, es_b, h, acc16, w1a, w1b, gcnw)


def _kmid_body(y_ref, tc_ref, agg_ref, acc_ref, b1_ref, w2_ref, b2_ref,
               gcnb_ref, w1a_ref, w1b_ref, gcnw_ref, yo_ref, tco_ref):
    hx, s = _layer_post(y_ref[...], tc_ref[...], agg_ref, acc_ref,
                        b1_ref, w2_ref, b2_ref, gcnb_ref)
    yo_ref[...], tco_ref[...] = _layer_pre(hx, s, acc_ref,
                                           w1a_ref, w1b_ref, gcnw_ref)


def _kmid(y, tcs, agg, acc16, b1, w2, b2, gcnb, w1a, w1b, gcnw):
    full = lambda shp: pl.BlockSpec(shp, lambda i: (0, 0))
    row = lambda w: pl.BlockSpec((B, w), lambda i: (i, 0))
    pair = lambda w: pl.BlockSpec((NC, B, w), lambda i: (0, i, 0))
    return pl.pallas_call(
        _kmid_body,
        grid=(GRID,),
        in_specs=[row(128), row(128), pair(128), pair(16), full((1, 128)),
                  full((128, 128)), full((1, 128)), full((1, 128)),
                  full((128, 128)), full((128, 128)), full((128, 128))],
        out_specs=[row(128), row(128)],
        out_shape=[jax.ShapeDtypeStruct((NP_, 128), jnp.float32),
                   jax.ShapeDtypeStruct((NP_, 128), jnp.float32)],
    )(y, tcs, agg, acc16, b1, w2, b2, gcnb, w1a, w1b, gcnw)


def _kfin_body(y_ref, tc_ref, agg_ref, acc_ref, b1_ref, w2_ref, b2_ref,
               gcnb_ref, batch_ref, wa_ref, wb_ref, whpb_ref, out_ref):
    i = pl.program_id(0)
    hx, s = _layer_post(y_ref[...], tc_ref[...], agg_ref, acc_ref,
                        b1_ref, w2_ref, b2_ref, gcnb_ref)
    hx2 = _dot(hx, wa_ref[...]) + _dot(s, wb_ref[...]) + whpb_ref[...]
    gids = lax.broadcasted_iota(jnp.int32, (NGRAPH, B), 0)
    mask = (gids == batch_ref[0]).astype(jnp.float32)
    part = _dot(mask, hx2)

    @pl.when(i == 0)
    def _():
        out_ref[...] = part

    @pl.when(i > 0)
    def _():
        out_ref[...] = out_ref[...] + part


def _kfin(y, tcs, agg, acc16, b1, w2, b2, gcnb, batch2d, wa, wb, whp_b):
    full = lambda shp: pl.BlockSpec(shp, lambda i: (0, 0))
    row = lambda w: pl.BlockSpec((B, w), lambda i: (i, 0))
    pair = lambda w: pl.BlockSpec((NC, B, w), lambda i: (0, i, 0))
    return pl.pallas_call(
        _kfin_body,
        grid=(GRID,),
        in_specs=[row(128), row(128), pair(128), pair(16), full((1, 128)),
                  full((128, 128)), full((1, 128)), full((1, 128)),
                  pl.BlockSpec((1, 1, B), lambda i: (i, 0, 0)),
                  full((128, 128)), full((128, 128)), full((1, 128))],
        out_specs=full((NGRAPH, 128)),
        out_shape=jax.ShapeDtypeStruct((NGRAPH, 128), jnp.float32),
    )(y, tcs, agg, acc16, b1, w2, b2, gcnb, batch2d, wa, wb, whp_b)


def _tc5_body(pool_ref, pw_ref, pb_ref, rw_ref, rb_ref, out_ref):
    p = jax.nn.relu(_dot(pool_ref[...], pw_ref[...]) + pb_ref[...])
    lg = _dot(p, rw_ref[...]) + rb_ref[...]
    m = jnp.max(lg, axis=1, keepdims=True)
    e = lg - m
    out_ref[...] = e - jnp.log(jnp.sum(jnp.exp(e), axis=1, keepdims=True))


def _tc5(pooled, post_w, post_b, ro_w, ro_b):
    full = lambda shp: pl.BlockSpec(shp, lambda i: (0, 0))
    return pl.pallas_call(
        _tc5_body,
        grid=(1,),
        in_specs=[full((NGRAPH, 128)), full((128, 128)), full((1, 128)),
                  full((128, NCLASS)), full((1, NCLASS))],
        out_specs=full((NGRAPH, NCLASS)),
        out_shape=jax.ShapeDtypeStruct((NGRAPH, NCLASS), jnp.float32),
    )(pooled, post_w, post_b, ro_w, ro_b)


def kernel(x, stc_enc, dists_max, W_pre, b_pre, dc_w1, dc_b1, dc_w2, dc_b2,
           hid_w, hid_b, pos_w, pos_b, es_w, es_b, gin_w1, gin_b1, gin_w2,
           gin_b2, gcn_w, gcn_b, whp_w, whp_b, post_w, post_b, ro_w, ro_b,
           edge_index, batch, dists_argmax):
    f32 = jnp.float32
    pad = NP_ - N
    xp = jnp.pad(x, ((0, pad), (0, 0)))
    dmp = jnp.pad(dists_max, ((0, pad), (0, 0)))
    stcp = jnp.pad(stc_enc, ((0, pad), (0, 0)))
    batch2d = jnp.pad(batch.astype(jnp.int32), (0, pad),
                      constant_values=NGRAPH).reshape(GRID, 1, B)
    src = edge_index[0].astype(jnp.int32)
    dst = edge_index[1].astype(jnp.int32)
    src_p = jnp.pad(src, (0, EP + 4 * ECH - E))
    dst_p = jnp.pad(dst, (0, EP + 4 * ECH - E), constant_values=NP_ - 1)
    aidx = jnp.pad(dists_argmax.reshape(-1).astype(jnp.int32),
                   (0, NAK + _GNB * GCH - N * K))

    ones_ch = jnp.ones((ECH, 16), f32)
    zeros16 = jnp.zeros((ROWS, 16), f32)
    zeros128 = jnp.zeros((ROWS, NHID), f32)

    acc16 = _deg_kernel(dst_p, ones_ch, zeros16)

    h, g, c, d = _tc1(xp, dmp, W_pre, b_pre.reshape(1, -1),
                      hid_w[:NHID], hid_w[NHID:], hid_b.reshape(1, -1),
                      dc_w1.reshape(1, NPG), dc_b1.reshape(1, NPG),
                      dc_w2.reshape(1, NPG), dc_b2.reshape(1, 1))

    sub = _gather_kernel(g, aidx)
    sub2d = sub.reshape(NP_, K * NPG)

    y, tcs = _kinit(sub2d, d, c, stcp, es_w[:SDIM], es_w[SDIM:],
                    es_b.reshape(1, -1), h, acc16,
                    gin_w1[0, :NHID], gin_w1[0, NHID:], gcn_w[0])

    nl = gin_w1.shape[0]
    for i in range(nl - 1):
        agg = _agg_kernel(y, tcs, src_p, dst_p, zeros128)
        y, tcs = _kmid(y, tcs, agg, acc16, gin_b1[i].reshape(1, -1),
                       gin_w2[i], gin_b2[i].reshape(1, -1),
                       gcn_b[i].reshape(1, -1),
                       gin_w1[i + 1, :NHID], gin_w1[i + 1, NHID:],
                       gcn_w[i + 1])

    agg = _agg_kernel(y, tcs, src_p, dst_p, zeros128)
    pooled = _kfin(y, tcs, agg, acc16, gin_b1[nl - 1].reshape(1, -1),
                   gin_w2[nl - 1], gin_b2[nl - 1].reshape(1, -1),
                   gcn_b[nl - 1].reshape(1, -1), batch2d,
                   whp_w[:NHID], whp_w[NHID:], whp_b.reshape(1, -1))
    return _tc5(pooled, post_w, post_b.reshape(1, -1), ro_w,
                ro_b.reshape(1, -1))


# speedup vs baseline: 1.7753x; 1.0004x over previous
import jax
import jax.numpy as jnp
from jax import lax
from jax.experimental import pallas as pl
from jax.experimental.pallas import tpu as pltpu
from jax.experimental.pallas import tpu_sc as plsc

N = 10000
E = 320000
K = 32
NHID = 128
SDIM = 32
NPG = 16
NCLASS = 16
NGRAPH = 128
NP_ = 10240
NAK = NP_ * K
NC, NS = 2, 16
ROWS = NP_ // NS
ECH = 120
TCHUNK = 168
EPT = TCHUNK * ECH
EP = NS * EPT
DPT = EP // NC // NS
DCHUNK = DPT // ECH
GPT = NAK // (NC * NS)
GCH = 128
_P = lax.Precision.HIGHEST
_MESH = plsc.VectorSubcoreMesh(core_axis_name="c", subcore_axis_name="s",
                               num_cores=NC, num_subcores=NS)
_SC_LINEAR = pltpu.CompilerParams(use_tc_tiling_on_sc=False)


def _dot(a, b):
    return jnp.dot(a, b, preferred_element_type=jnp.float32, precision=_P)


_DNB = 4


def _deg_body(dst_hbm, ones_hbm, zeros_hbm, out_hbm, *scr):
    ones_v = scr[0]
    didx = scr[1:1 + _DNB]
    isem = scr[1 + _DNB:1 + 2 * _DNB]
    acc_sh = scr[1 + 2 * _DNB]
    c = lax.axis_index("c")
    s = lax.axis_index("s")
    base0 = c * (EP // NC) + s * DPT

    def iload(g, k):
        pltpu.async_copy(dst_hbm.at[pl.ds(base0 + g * ECH, ECH)],
                         didx[k], isem[k])

    def iwait(g, k):
        pltpu.make_async_copy(dst_hbm.at[pl.ds(base0 + g * ECH, ECH)],
                              didx[k], isem[k]).wait()

    pltpu.sync_copy(ones_hbm, ones_v)
    for k in range(_DNB):
        iload(k, k)
    pltpu.sync_copy(zeros_hbm, acc_sh.at[pl.ds(s * ROWS, ROWS)])
    plsc.subcore_barrier()

    def ring(i, carry):
        g = _DNB * i
        for k in range(_DNB):
            iwait(g + k, k)
            pltpu.sync_copy(ones_v, acc_sh.at[didx[k]], add=True)
            iload(g + k + _DNB, k)
        return carry

    lax.fori_loop(0, DCHUNK // _DNB, ring, 0)
    for k in range(_DNB):
        iwait(DCHUNK + k, k)
    plsc.subcore_barrier()
    pltpu.sync_copy(acc_sh.at[pl.ds(s * ROWS, ROWS)],
                    out_hbm.at[c, pl.ds(s * ROWS, ROWS)])


_deg_kernel = pl.kernel(
    _deg_body,
    out_type=jax.ShapeDtypeStruct((NC, NP_, 16), jnp.float32),
    mesh=_MESH,
    scratch_types=(
        [pltpu.VMEM((ECH, 16), jnp.float32)]
        + [pltpu.VMEM((ECH,), jnp.int32)] * _DNB
        + [pltpu.SemaphoreType.DMA] * _DNB
        + [pltpu.VMEM_SHARED((NP_, 16), jnp.float32)]
    ),
    compiler_params=_SC_LINEAR,
)


_GNB = 4


def _gather_body(g_hbm, idx_hbm, out_hbm, *scr):
    idxs = scr[0:_GNB]
    bufs = scr[_GNB:2 * _GNB]
    gsem = scr[2 * _GNB:3 * _GNB]
    isem = scr[3 * _GNB:4 * _GNB]
    wsem = scr[4 * _GNB:5 * _GNB]
    wid = lax.axis_index("s") * NC + lax.axis_index("c")
    base0 = wid * GPT

    def iload(g, k):
        pltpu.async_copy(idx_hbm.at[pl.ds(base0 + g * GCH, GCH)],
                         idxs[k], isem[k])

    def iwait(g, k):
        pltpu.make_async_copy(idx_hbm.at[pl.ds(base0 + g * GCH, GCH)],
                              idxs[k], isem[k]).wait()

    def gstart(k):
        pltpu.async_copy(g_hbm.at[idxs[k]], bufs[k], gsem[k])

    def gwait(k):
        pltpu.make_async_copy(g_hbm.at[idxs[k]], bufs[k], gsem[k]).wait()

    def wstart(g, k):
        pltpu.async_copy(bufs[k], out_hbm.at[pl.ds(base0 + g * GCH, GCH)],
                         wsem[k])

    def wwait(g, k):
        pltpu.make_async_copy(bufs[k],
                              out_hbm.at[pl.ds(base0 + g * GCH, GCH)],
                              wsem[k]).wait()

    for k in range(_GNB):
        iload(k, k)
    for k in range(_GNB):
        iwait(k, k)
        gstart(k)

    def ring(i, carry):
        g = _GNB * i
        for k in range(_GNB):
            gwait(k)
            wstart(g + k, k)
            iload(g + k + _GNB, k)
            iwait(g + k + _GNB, k)
            wwait(g + k, k)
            gstart(k)
        return carry

    lax.fori_loop(0, GPT // GCH // _GNB, ring, 0)
    for k in range(_GNB):
        gwait(k)


_gather_kernel = pl.kernel(
    _gather_body,
    out_type=jax.ShapeDtypeStruct((NAK, 16), jnp.float32),
    mesh=_MESH,
    scratch_types=(
        [pltpu.VMEM((GCH,), jnp.int32)] * _GNB
        + [pltpu.VMEM((GCH, 16), jnp.float32)] * _GNB
        + [pltpu.SemaphoreType.DMA] * (3 * _GNB)
    ),
    compiler_params=_SC_LINEAR,
)


_ANB = 3


def _agg_body(y_hbm, tc_hbm, src_hbm, dst_hbm, zeros_hbm, out_hbm, *scr):
    sidx = scr[0:_ANB]
    didx = scr[_ANB:2 * _ANB]
    bufs = scr[2 * _ANB:3 * _ANB]
    acc_sh = scr[3 * _ANB]
    gsem = scr[3 * _ANB + 1:4 * _ANB + 1]
    isem = scr[4 * _ANB + 1:5 * _ANB + 1]
    c = lax.axis_index("c")
    s = lax.axis_index("s")
    base0 = s * EPT

    def iload(g, k):
        pltpu.async_copy(src_hbm.at[pl.ds(base0 + g * ECH, ECH)],
                         sidx[k], isem[k])
        pltpu.async_copy(dst_hbm.at[pl.ds(base0 + g * ECH, ECH)],
                         didx[k], isem[k])

    def iwait(g, k):
        pltpu.make_async_copy(
            src_hbm.at[pl.ds(base0 + g * ECH, ECH)], sidx[k], isem[k]).wait()
        pltpu.make_async_copy(
            dst_hbm.at[pl.ds(base0 + g * ECH, ECH)], didx[k], isem[k]).wait()

    def gstart(k):
        @pl.when(c == 0)
        def _():
            pltpu.async_copy(y_hbm.at[sidx[k]], bufs[k], gsem[k])

        @pl.when(c == 1)
        def _():
            pltpu.async_copy(tc_hbm.at[sidx[k]], bufs[k], gsem[k])

    def gwait(k):
        pltpu.make_async_copy(y_hbm.at[sidx[k]], bufs[k], gsem[k]).wait()

    def scat(k):
        pltpu.sync_copy(bufs[k], acc_sh.at[didx[k]], add=True)

    for k in range(_ANB):
        iload(k, k)
    pltpu.sync_copy(zeros_hbm, acc_sh.at[pl.ds(s * ROWS, ROWS)])
    plsc.subcore_barrier()
    for k in range(_ANB):
        iwait(k, k)
        gstart(k)

    def ring(i, carry):
        g = _ANB * i
        for k in range(_ANB):
            gwait(k)
            scat(k)
            iload(g + k + _ANB, k)
            iwait(g + k + _ANB, k)
            gstart(k)
        return carry

    lax.fori_loop(0, TCHUNK // _ANB, ring, 0)
    for k in range(_ANB):
        gwait(k)
    plsc.subcore_barrier()
    pltpu.sync_copy(acc_sh.at[pl.ds(s * ROWS, ROWS)],
                    out_hbm.at[c, pl.ds(s * ROWS, ROWS)])


_agg_kernel = pl.kernel(
    _agg_body,
    out_type=jax.ShapeDtypeStruct((NC, NP_, NHID), jnp.float32),
    mesh=_MESH,
    scratch_types=(
        [pltpu.VMEM((ECH,), jnp.int32)] * (2 * _ANB)
        + [pltpu.VMEM((ECH, NHID), jnp.float32)] * _ANB
        + [pltpu.VMEM_SHARED((NP_, NHID), jnp.float32)]
        + [pltpu.SemaphoreType.DMA] * (2 * _ANB)
    ),
)


B = 1024
GRID = NP_ // B


def _tc1_body(x_ref, dm_ref, wpre_ref, bpre_ref, wa_ref, wb_ref, hidb_ref,
              dc1_ref, db1_ref, dc2_ref, db2_ref,
              h_ref, g_ref, c_ref, d_ref):
    h = _dot(x_ref[...], wpre_ref[...]) + bpre_ref[...]
    h_ref[...] = h
    g_ref[...] = _dot(h, wa_ref[...])
    c_ref[...] = _dot(h, wb_ref[...]) + hidb_ref[...]
    dm = dm_ref[...]
    d = jnp.zeros_like(dm) + db2_ref[0, 0]
    for p in range(NPG):
        d = d + jax.nn.relu(dm * dc1_ref[0, p] + db1_ref[0, p]) * dc2_ref[0, p]
    d_ref[...] = d


def _tc1(xp, dmp, w_pre, b_pre, wa, wb, hid_b, dc_w1, dc_b1, dc_w2, dc_b2):
    full = lambda shp: pl.BlockSpec(shp, lambda i: (0, 0))
    smem = lambda shp: pl.BlockSpec(shp, lambda i: (0, 0),
                                    memory_space=pltpu.SMEM)
    row = lambda w: pl.BlockSpec((B, w), lambda i: (i, 0))
    return pl.pallas_call(
        _tc1_body,
        grid=(GRID,),
        in_specs=[row(128), row(K), full((128, 128)), full((1, 128)),
                  full((128, NPG)), full((128, NPG)), full((1, NPG)),
                  smem((1, NPG)), smem((1, NPG)), smem((1, NPG)),
                  smem((1, 1))],
        out_specs=[row(128), row(NPG), row(NPG), row(K)],
        out_shape=[jax.ShapeDtypeStruct((NP_, 128), jnp.float32),
                   jax.ShapeDtypeStruct((NP_, NPG), jnp.float32),
                   jax.ShapeDtypeStruct((NP_, NPG), jnp.float32),
                   jax.ShapeDtypeStruct((NP_, K), jnp.float32)],
    )(xp, dmp, w_pre, b_pre, wa, wb, hid_b, dc_w1, dc_b1, dc_w2, dc_b2)


def _layer_post(y, tc, agg_ref, acc_ref, b1_ref, w2_ref, b2_ref, gcnb_ref):
    deg = acc_ref[0, :, 0:1] + acc_ref[1, :, 0:1] + 1.0
    dis = lax.rsqrt(deg)
    hg = _dot(jax.nn.relu(y + agg_ref[0] + b1_ref[...]),
              w2_ref[...]) + b2_ref[...]
    hx = jax.nn.relu(hg)
    s = jnp.tanh(dis * (agg_ref[1] + tc) + gcnb_ref[...])
    return hx, s


def _layer_pre(hx, s, acc_ref, w1a_ref, w1b_ref, gcnw_ref):
    y = _dot(hx, w1a_ref[...]) + _dot(s, w1b_ref[...])
    hs = _dot(s, gcnw_ref[...])
    deg = acc_ref[0, :, 0:1] + acc_ref[1, :, 0:1] + 1.0
    dis = lax.rsqrt(deg)
    return y, dis * hs


def _kinit_body(sub_ref, d_ref, c_ref, stc_ref, eswa_ref, eswb_ref, esb_ref,
                h_ref, acc_ref, w1a_ref, w1b_ref, gcnw_ref, y_ref, tc_ref):
    kk = lax.broadcasted_iota(jnp.int32, (K, K * NPG), 0)
    mm = lax.broadcasted_iota(jnp.int32, (K, K * NPG), 1)
    e32 = (kk == mm // NPG).astype(jnp.float32)
    jj = lax.broadcasted_iota(jnp.int32, (NPG, K * NPG), 0)
    m2 = lax.broadcasted_iota(jnp.int32, (NPG, K * NPG), 1)
    e16 = (jj == m2 % NPG).astype(jnp.float32)
    d_exp = _dot(d_ref[...], e32)
    c_t = _dot(c_ref[...], e16)
    msgs = jax.nn.relu(d_exp * sub_ref[...] + c_t)
    x1 = _dot(msgs, e16.T) * (1.0 / K)
    s = (_dot(stc_ref[...], eswa_ref[...])
         + _dot(x1, eswb_ref[...]) + esb_ref[...])
    y_ref[...], tc_ref[...] = _layer_pre(h_ref[...], s, acc_ref,
                                         w1a_ref, w1b_ref, gcnw_ref)


def _kinit(sub2d, dp, cp, stcp, eswa, eswb, es_b, h, acc16, w1a, w1b, gcnw):
    full = lambda shp: pl.BlockSpec(shp, lambda i: (0, 0))
    row = lambda w: pl.BlockSpec((B, w), lambda i: (i, 0))
    pair = lambda w: pl.BlockSpec((NC, B, w), lambda i: (0, i, 0))
    return pl.pallas_call(
        _kinit_body,
        grid=(GRID,),
        in_specs=[row(K * NPG), row(K), row(NPG), row(SDIM),
                  full((SDIM, 128)), full((NPG, 128)), full((1, 128)),
                  row(128), pair(16),
                  full((128, 128)), full((128, 128)), full((128, 128))],
        out_specs=[row(128), row(128)],
        out_shape=[jax.ShapeDtypeStruct((NP_, 128), jnp.float32),
                   jax.ShapeDtypeStruct((NP_, 128), jnp.float32)],
    )(sub2d, dp, cp, stcp, eswa, eswb, es_b, h, acc16, w1a, w1b, gcnw)


def _kmid_body(y_ref, tc_ref, agg_ref, acc_ref, b1_ref, w2_ref, b2_ref,
               gcnb_ref, w1a_ref, w1b_ref, gcnw_ref, yo_ref, tco_ref):
    hx, s = _layer_post(y_ref[...], tc_ref[...], agg_ref, acc_ref,
                        b1_ref, w2_ref, b2_ref, gcnb_ref)
    yo_ref[...], tco_ref[...] = _layer_pre(hx, s, acc_ref,
                                           w1a_ref, w1b_ref, gcnw_ref)


def _kmid(y, tcs, agg, acc16, b1, w2, b2, gcnb, w1a, w1b, gcnw):
    full = lambda shp: pl.BlockSpec(shp, lambda i: (0, 0))
    row = lambda w: pl.BlockSpec((B, w), lambda i: (i, 0))
    pair = lambda w: pl.BlockSpec((NC, B, w), lambda i: (0, i, 0))
    return pl.pallas_call(
        _kmid_body,
        grid=(GRID,),
        in_specs=[row(128), row(128), pair(128), pair(16), full((1, 128)),
                  full((128, 128)), full((1, 128)), full((1, 128)),
                  full((128, 128)), full((128, 128)), full((128, 128))],
        out_specs=[row(128), row(128)],
        out_shape=[jax.ShapeDtypeStruct((NP_, 128), jnp.float32),
                   jax.ShapeDtypeStruct((NP_, 128), jnp.float32)],
    )(y, tcs, agg, acc16, b1, w2, b2, gcnb, w1a, w1b, gcnw)


def _kfin_body(y_ref, tc_ref, agg_ref, acc_ref, b1_ref, w2_ref, b2_ref,
               gcnb_ref, batch_ref, wa_ref, wb_ref, whpb_ref, out_ref):
    i = pl.program_id(0)
    hx, s = _layer_post(y_ref[...], tc_ref[...], agg_ref, acc_ref,
                        b1_ref, w2_ref, b2_ref, gcnb_ref)
    hx2 = _dot(hx, wa_ref[...]) + _dot(s, wb_ref[...]) + whpb_ref[...]
    gids = lax.broadcasted_iota(jnp.int32, (NGRAPH, B), 0)
    mask = (gids == batch_ref[0]).astype(jnp.float32)
    part = _dot(mask, hx2)

    @pl.when(i == 0)
    def _():
        out_ref[...] = part

    @pl.when(i > 0)
    def _():
        out_ref[...] = out_ref[...] + part


def _kfin(y, tcs, agg, acc16, b1, w2, b2, gcnb, batch2d, wa, wb, whp_b):
    full = lambda shp: pl.BlockSpec(shp, lambda i: (0, 0))
    row = lambda w: pl.BlockSpec((B, w), lambda i: (i, 0))
    pair = lambda w: pl.BlockSpec((NC, B, w), lambda i: (0, i, 0))
    return pl.pallas_call(
        _kfin_body,
        grid=(GRID,),
        in_specs=[row(128), row(128), pair(128), pair(16), full((1, 128)),
                  full((128, 128)), full((1, 128)), full((1, 128)),
                  pl.BlockSpec((1, 1, B), lambda i: (i, 0, 0)),
                  full((128, 128)), full((128, 128)), full((1, 128))],
        out_specs=full((NGRAPH, 128)),
        out_shape=jax.ShapeDtypeStruct((NGRAPH, 128), jnp.float32),
    )(y, tcs, agg, acc16, b1, w2, b2, gcnb, batch2d, wa, wb, whp_b)


def _tc5_body(pool_ref, pw_ref, pb_ref, rw_ref, rb_ref, out_ref):
    p = jax.nn.relu(_dot(pool_ref[...], pw_ref[...]) + pb_ref[...])
    lg = _dot(p, rw_ref[...]) + rb_ref[...]
    m = jnp.max(lg, axis=1, keepdims=True)
    e = lg - m
    out_ref[...] = e - jnp.log(jnp.sum(jnp.exp(e), axis=1, keepdims=True))


def _tc5(pooled, post_w, post_b, ro_w, ro_b):
    full = lambda shp: pl.BlockSpec(shp, lambda i: (0, 0))
    return pl.pallas_call(
        _tc5_body,
        grid=(1,),
        in_specs=[full((NGRAPH, 128)), full((128, 128)), full((1, 128)),
                  full((128, NCLASS)), full((1, NCLASS))],
        out_specs=full((NGRAPH, NCLASS)),
        out_shape=jax.ShapeDtypeStruct((NGRAPH, NCLASS), jnp.float32),
    )(pooled, post_w, post_b, ro_w, ro_b)


def kernel(x, stc_enc, dists_max, W_pre, b_pre, dc_w1, dc_b1, dc_w2, dc_b2,
           hid_w, hid_b, pos_w, pos_b, es_w, es_b, gin_w1, gin_b1, gin_w2,
           gin_b2, gcn_w, gcn_b, whp_w, whp_b, post_w, post_b, ro_w, ro_b,
           edge_index, batch, dists_argmax):
    f32 = jnp.float32
    pad = NP_ - N
    xp = jnp.pad(x, ((0, pad), (0, 0)))
    dmp = jnp.pad(dists_max, ((0, pad), (0, 0)))
    stcp = jnp.pad(stc_enc, ((0, pad), (0, 0)))
    batch2d = jnp.pad(batch.astype(jnp.int32), (0, pad),
                      constant_values=NGRAPH).reshape(GRID, 1, B)
    src = edge_index[0].astype(jnp.int32)
    dst = edge_index[1].astype(jnp.int32)
    src_p = jnp.pad(src, (0, EP + 4 * ECH - E))
    dst_p = jnp.pad(dst, (0, EP + 4 * ECH - E), constant_values=NP_ - 1)
    aidx = jnp.pad(dists_argmax.reshape(-1).astype(jnp.int32),
                   (0, NAK + _GNB * GCH - N * K))

    ones_ch = jnp.ones((ECH, 16), f32)
    zeros16 = jnp.zeros((ROWS, 16), f32)
    zeros128 = jnp.zeros((ROWS, NHID), f32)

    acc16 = _deg_kernel(dst_p, ones_ch, zeros16)

    h, g, c, d = _tc1(xp, dmp, W_pre, b_pre.reshape(1, -1),
                      hid_w[:NHID], hid_w[NHID:], hid_b.reshape(1, -1),
                      dc_w1.reshape(1, NPG), dc_b1.reshape(1, NPG),
                      dc_w2.reshape(1, NPG), dc_b2.reshape(1, 1))

    sub = _gather_kernel(g, aidx)
    sub2d = sub.reshape(NP_, K * NPG)

    y, tcs = _kinit(sub2d, d, c, stcp, es_w[:SDIM], es_w[SDIM:],
                    es_b.reshape(1, -1), h, acc16,
                    gin_w1[0, :NHID], gin_w1[0, NHID:], gcn_w[0])

    nl = gin_w1.shape[0]
    for i in range(nl - 1):
        agg = _agg_kernel(y, tcs, src_p, dst_p, zeros128)
        y, tcs = _kmid(y, tcs, agg, acc16, gin_b1[i].reshape(1, -1),
                       gin_w2[i], gin_b2[i].reshape(1, -1),
                       gcn_b[i].reshape(1, -1),
                       gin_w1[i + 1, :NHID], gin_w1[i + 1, NHID:],
                       gcn_w[i + 1])

    agg = _agg_kernel(y, tcs, src_p, dst_p, zeros128)
    pooled = _kfin(y, tcs, agg, acc16, gin_b1[nl - 1].reshape(1, -1),
                   gin_w2[nl - 1], gin_b2[nl - 1].reshape(1, -1),
                   gcn_b[nl - 1].reshape(1, -1), batch2d,
                   whp_w[:NHID], whp_w[NHID:], whp_b.reshape(1, -1))
    return _tc5(pooled, post_w, post_b.reshape(1, -1), ro_w,
                ro_b.reshape(1, -1))
